# Initial kernel scaffold; baseline (speedup 1.0000x reference)
import jax
import jax.numpy as jnp
from jax.experimental import pallas as pl


def kernel(h, edge_index, coord, edge_attr, We1, be1, We2, be2, Wn1, bn1, Wn2, bn2, Wc1, bc1, Wc2, bc2):
    raise NotImplementedError("write your pallas kernel here")


# speedup vs baseline: 3.0078x; 3.0078x over previous
import dataclasses
import functools

import jax
import jax.numpy as jnp
from jax import lax
from jax.experimental import pallas as pl
from jax.experimental.pallas import tpu as pltpu
from jax.experimental.pallas import tpu_sc as plsc

NC = 2
NS = 16
NW = NC * NS
CHUNK = 128


def _sc_compiler_params():
    cp = pltpu.CompilerParams()
    if "needs_layout_passes" in pltpu.CompilerParams.__dataclass_fields__:
        cp = dataclasses.replace(cp, needs_layout_passes=False)
    return cp


def _silu(x):
    return x * jax.nn.sigmoid(x)


def _build_tables(h, We1a, We1b, be1r):
    N, D = h.shape

    def body(h_ref, wa_ref, wb_ref, b1_ref, p1_ref, p2_ref):
        hh = h_ref[...]
        p1_ref[...] = jnp.dot(hh, wa_ref[...],
                              preferred_element_type=jnp.float32)
        p2_ref[...] = jnp.dot(hh, wb_ref[...],
                              preferred_element_type=jnp.float32) + b1_ref[...]

    return pl.pallas_call(
        body,
        out_shape=(
            jax.ShapeDtypeStruct((N, D), jnp.float32),
            jax.ShapeDtypeStruct((N, D), jnp.float32),
        ),
    )(h, We1a, We1b, be1r)


def _sc_gather(p1, p2, c4flat, rows2d, cols2d, e_pad):
    N, D = p1.shape
    n4 = c4flat.shape[0]
    ept = e_pad // NW
    ch = ept // CHUNK
    mesh = plsc.VectorSubcoreMesh(core_axis_name="c", subcore_axis_name="s")

    @functools.partial(
        pl.kernel,
        out_type=(
            jax.ShapeDtypeStruct((e_pad, D), jnp.float32),
            jax.ShapeDtypeStruct((e_pad, D), jnp.float32),
            jax.ShapeDtypeStruct((e_pad * 16,), jnp.float32),
        ),
        mesh=mesh,
        scratch_types=[
            pltpu.VMEM((ch, CHUNK), jnp.int32),
            pltpu.VMEM((ch, CHUNK), jnp.int32),
            pltpu.VMEM((CHUNK, D), jnp.float32),
            pltpu.VMEM((CHUNK, D), jnp.float32),
            pltpu.VMEM((n4,), jnp.float32),
            pltpu.VMEM((CHUNK * 16,), jnp.float32),
            pltpu.SemaphoreType.DMA,
            pltpu.SemaphoreType.DMA,
        ],
        compiler_params=_sc_compiler_params(),
    )
    def k(p1_hbm, p2_hbm, c4_hbm, ri_hbm, ci_hbm, g1_hbm, g2_hbm, cd_hbm,
          ir_v, ic_v, b1_v, b2_v, cl_v, cdb_v, sem1, sem2):
        wid = lax.axis_index("c") * NS + lax.axis_index("s")
        pltpu.sync_copy(ri_hbm.at[pl.ds(wid * ch, ch)], ir_v)
        pltpu.sync_copy(ci_hbm.at[pl.ds(wid * ch, ch)], ic_v)
        pltpu.sync_copy(c4_hbm, cl_v)

        zero16 = jnp.zeros((16,), jnp.float32)

        @pl.loop(0, CHUNK * 16, step=16)
        def _(i):
            cdb_v[pl.ds(i, 16)] = zero16

        iota16 = lax.iota(jnp.int32, 16)
        base = wid * ept

        @pl.loop(0, ch)
        def _(j):
            dst = pl.ds(base + j * CHUNK, CHUNK)
            cp1 = pltpu.async_copy(p1_hbm.at[ir_v.at[j]], b1_v, sem1)
            cp2 = pltpu.async_copy(p2_hbm.at[ic_v.at[j]], b2_v, sem2)
            for sub in range(CHUNK // 16):
                er = ir_v[j, pl.ds(sub * 16, 16)]
                ec = ic_v[j, pl.ds(sub * 16, 16)]
                pos = sub * 256 + iota16 * 16
                for c in range(3):
                    xr = plsc.load_gather(cl_v, [er * 4 + c])
                    xc = plsc.load_gather(cl_v, [ec * 4 + c])
                    plsc.store_scatter(cdb_v, [pos + c], xr - xc)
            cp1.wait()
            pltpu.sync_copy(b1_v, g1_hbm.at[dst])
            cp2.wait()
            pltpu.sync_copy(b2_v, g2_hbm.at[dst])
            pltpu.sync_copy(
                cdb_v, cd_hbm.at[pl.ds((base + j * CHUNK) * 16, CHUNK * 16)])

    return k(p1, p2, c4flat, rows2d, cols2d)


def _edge_mlp(g1, g2, cd, w1r, We2, be2r, Wc1, bc1r, wc2r, bc2r, n_edges,
              blk):
    e_pad, D = g1.shape
    grid = (e_pad // blk,)

    def body(g1_ref, g2_ref, cd_ref, w1_ref, w2_ref, b2_ref, wc1_ref,
             bc1_ref, wc2_ref, bc2_ref, ef_ref, tr_ref):
        s = g1_ref[...] + g2_ref[...]
        cdv = cd_ref[...]
        radial = jnp.sum(cdv * cdv, axis=1, keepdims=True)
        ef = _silu(s + radial * w1_ref[...])
        edge_feat = _silu(
            jnp.dot(ef, w2_ref[...], preferred_element_type=jnp.float32)
            + b2_ref[...])
        tt = _silu(
            jnp.dot(edge_feat, wc1_ref[...], preferred_element_type=jnp.float32)
            + bc1_ref[...])
        t = jnp.sum(tt * wc2_ref[...], axis=1, keepdims=True) + bc2_ref[...]
        scale = t / jnp.sqrt(radial + 1e-8)
        trans = cdv * scale
        eid = pl.program_id(0) * blk + lax.broadcasted_iota(
            jnp.int32, (blk, 1), 0)
        valid = eid < n_edges
        ef_ref[...] = jnp.where(valid, edge_feat, 0.0)
        tr_ref[...] = jnp.where(valid, trans, 0.0)

    const = pl.BlockSpec((1, D), lambda i: (0, 0))
    return pl.pallas_call(
        body,
        grid=grid,
        in_specs=[
            pl.BlockSpec((blk, D), lambda i: (i, 0)),
            pl.BlockSpec((blk, D), lambda i: (i, 0)),
            pl.BlockSpec((blk, 16), lambda i: (i, 0)),
            const,
            pl.BlockSpec((D, D), lambda i: (0, 0)),
            const,
            pl.BlockSpec((D, D), lambda i: (0, 0)),
            const,
            const,
            pl.BlockSpec((1, 1), lambda i: (0, 0)),
        ],
        out_specs=[
            pl.BlockSpec((blk, D), lambda i: (i, 0)),
            pl.BlockSpec((blk, 16), lambda i: (i, 0)),
        ],
        out_shape=(
            jax.ShapeDtypeStruct((e_pad, D), jnp.float32),
            jax.ShapeDtypeStruct((e_pad, 16), jnp.float32),
        ),
        compiler_params=pltpu.CompilerParams(
            dimension_semantics=("parallel",)),
    )(g1, g2, cd, w1r, We2, be2r, Wc1, bc1r, wc2r, bc2r)


def _sc_segment_sum_h(ef, rows2d, zh, npad):
    e_pad, D = ef.shape
    ept = e_pad // NW
    ch = ept // CHUNK
    npt = npad // NS
    mesh = plsc.VectorSubcoreMesh(core_axis_name="c", subcore_axis_name="s")

    @functools.partial(
        pl.kernel,
        out_type=jax.ShapeDtypeStruct((NC * npad, D), jnp.float32),
        mesh=mesh,
        scratch_types=[
            pltpu.VMEM_SHARED((npad, D), jnp.float32),
            pltpu.VMEM((ch, CHUNK), jnp.int32),
            pltpu.VMEM((CHUNK, D), jnp.float32),
        ],
        compiler_params=_sc_compiler_params(),
    )
    def k(ef_hbm, ri_hbm, zh_hbm, ph_hbm, acc_h, idx_v, vh_v):
        cid = lax.axis_index("c")
        sid = lax.axis_index("s")
        wid = cid * NS + sid
        nslc = pl.ds(sid * npt, npt)
        pltpu.sync_copy(zh_hbm, acc_h.at[nslc])
        pltpu.sync_copy(ri_hbm.at[pl.ds(wid * ch, ch)], idx_v)
        plsc.subcore_barrier()
        base = wid * ept

        @pl.loop(0, ch)
        def _(j):
            pltpu.sync_copy(ef_hbm.at[pl.ds(base + j * CHUNK, CHUNK)], vh_v)
            pltpu.sync_copy(vh_v, acc_h.at[idx_v.at[j]], add=True)

        plsc.subcore_barrier()
        pltpu.sync_copy(acc_h.at[nslc],
                        ph_hbm.at[pl.ds(cid * npad + sid * npt, npt)])

    return k(ef, rows2d, zh)


def _sc_segment_sum_c(trflat, rows2d, zc, npad):
    e_pad16 = trflat.shape[0]
    e_pad = e_pad16 // 16
    ept = e_pad // NW
    ch = ept // CHUNK
    n4 = npad * 4
    mesh = plsc.VectorSubcoreMesh(core_axis_name="c", subcore_axis_name="s")

    @functools.partial(
        pl.kernel,
        out_type=jax.ShapeDtypeStruct((NW * n4,), jnp.float32),
        mesh=mesh,
        scratch_types=[
            pltpu.VMEM((n4,), jnp.float32),
            pltpu.VMEM((ch, CHUNK), jnp.int32),
            pltpu.VMEM((CHUNK * 16,), jnp.float32),
        ],
        compiler_params=_sc_compiler_params(),
    )
    def k(tr_hbm, ri_hbm, zc_hbm, pc_hbm, acc_c, idx_v, vt_v):
        wid = lax.axis_index("c") * NS + lax.axis_index("s")
        pltpu.sync_copy(zc_hbm, acc_c)
        pltpu.sync_copy(ri_hbm.at[pl.ds(wid * ch, ch)], idx_v)
        iota16 = lax.iota(jnp.int32, 16)
        base = wid * ept

        @pl.loop(0, ch)
        def _(j):
            pltpu.sync_copy(
                tr_hbm.at[pl.ds((base + j * CHUNK) * 16, CHUNK * 16)], vt_v)
            for sub in range(CHUNK // 16):
                en = idx_v[j, pl.ds(sub * 16, 16)]
                pos = sub * 256 + iota16 * 16
                for c in range(3):
                    v = plsc.load_gather(vt_v, [pos + c])
                    plsc.addupdate_scatter(acc_c, [en * 4 + c], v)

        pltpu.sync_copy(acc_c, pc_hbm.at[pl.ds(wid * n4, n4)])

    return k(trflat, rows2d, zc)


def _node_mlp(h, c4mat, ph, pcmat, Wn1a, Wn1b, bn1r, Wn2, bn2r, npad):
    N, D = h.shape
    rows4 = c4mat.shape[0]

    def body(h_ref, c_ref, ph_ref, pc_ref, wa_ref, wb_ref, b1_ref,
             w2_ref, b2_ref, ho_ref, co_ref):
        hh = h_ref[...]
        agg = ph_ref[:N, :] + ph_ref[npad:npad + N, :]
        m1 = _silu(
            jnp.dot(hh, wa_ref[...], preferred_element_type=jnp.float32)
            + jnp.dot(agg, wb_ref[...], preferred_element_type=jnp.float32)
            + b1_ref[...])
        m = jnp.dot(m1, w2_ref[...], preferred_element_type=jnp.float32)
        ho_ref[...] = hh + m + b2_ref[...]
        co_ref[...] = c_ref[...] + jnp.sum(pc_ref[...], axis=0)

    return pl.pallas_call(
        body,
        out_shape=(
            jax.ShapeDtypeStruct((N, D), jnp.float32),
            jax.ShapeDtypeStruct((rows4, 128), jnp.float32),
        ),
    )(h, c4mat, ph, pcmat, Wn1a, Wn1b, bn1r, Wn2, bn2r)


def kernel(h, edge_index, coord, edge_attr,
           We1, be1, We2, be2, Wn1, bn1, Wn2, bn2, Wc1, bc1, Wc2, bc2):
    del edge_attr
    N, D = h.shape
    E = edge_index.shape[1]
    tile_edges = NW * CHUNK * 8
    e_pad = ((E + tile_edges - 1) // tile_edges) * tile_edges
    npad = ((N + NS * 8 - 1) // (NS * 8)) * (NS * 8)

    row = edge_index[0].astype(jnp.int32)
    col = edge_index[1].astype(jnp.int32)
    rows2d = jnp.pad(row, (0, e_pad - E)).reshape(e_pad // CHUNK, CHUNK)
    cols2d = jnp.pad(col, (0, e_pad - E)).reshape(e_pad // CHUNK, CHUNK)

    c4flat = jnp.pad(coord, ((0, npad - N), (0, 1))).reshape(-1)
    We1a = We1[:D]
    We1b = We1[D:2 * D]
    w1r = We1[2 * D].reshape(1, D)
    be1r = be1.reshape(1, D)
    be2r = be2.reshape(1, D)
    bc1r = bc1.reshape(1, D)
    wc2r = Wc2.reshape(1, D)
    bc2r = bc2.reshape(1, 1)
    bn1r = bn1.reshape(1, D)
    bn2r = bn2.reshape(1, D)
    Wn1a = Wn1[:D]
    Wn1b = Wn1[D:]

    p1, p2 = _build_tables(h, We1a, We1b, be1r)
    g1, g2, cdflat = _sc_gather(p1, p2, c4flat, rows2d, cols2d, e_pad)
    cd = cdflat.reshape(e_pad, 16)
    ef, tr = _edge_mlp(g1, g2, cd, w1r, We2, be2r, Wc1, bc1r, wc2r, bc2r,
                       E, 2048)
    zh = jnp.zeros((npad // NS, D), jnp.float32)
    zc = jnp.zeros((npad * 4,), jnp.float32)
    ph = _sc_segment_sum_h(ef, rows2d, zh, npad)
    pc = _sc_segment_sum_c(tr.reshape(-1), rows2d, zc, npad)
    rows4 = npad * 4 // 128
    pcmat = pc.reshape(NW, rows4, 128)
    c4mat = c4flat.reshape(rows4, 128)
    h_out, co_mat = _node_mlp(h, c4mat, ph, pcmat, Wn1a, Wn1b, bn1r,
                              Wn2, bn2r, npad)
    coord_out = co_mat.reshape(npad, 4)[:N, :3]
    return (h_out, coord_out)


# speedup vs baseline: 3.3935x; 1.1282x over previous
import dataclasses
import functools

import jax
import jax.numpy as jnp
from jax import lax
from jax.experimental import pallas as pl
from jax.experimental.pallas import tpu as pltpu
from jax.experimental.pallas import tpu_sc as plsc

NC = 2
NS = 16
NW = NC * NS
CHUNK = 128


def _sc_compiler_params():
    cp = pltpu.CompilerParams()
    if "needs_layout_passes" in pltpu.CompilerParams.__dataclass_fields__:
        cp = dataclasses.replace(cp, needs_layout_passes=False)
    return cp


def _silu(x):
    return x * jax.nn.sigmoid(x)


def _build_tables(h, We1a, We1b, be1r):
    N, D = h.shape

    def body(h_ref, wa_ref, wb_ref, b1_ref, p1_ref, p2_ref):
        hh = h_ref[...]
        p1_ref[...] = jnp.dot(hh, wa_ref[...],
                              preferred_element_type=jnp.float32)
        p2_ref[...] = jnp.dot(hh, wb_ref[...],
                              preferred_element_type=jnp.float32) + b1_ref[...]

    return pl.pallas_call(
        body,
        out_shape=(
            jax.ShapeDtypeStruct((N, D), jnp.float32),
            jax.ShapeDtypeStruct((N, D), jnp.float32),
        ),
    )(h, We1a, We1b, be1r)


def _sc_gather(p1, p2, c4flat, rows2d, cols2d, e_pad):
    N, D = p1.shape
    n4 = c4flat.shape[0]
    ept = e_pad // NW
    ch = ept // CHUNK
    mesh = plsc.VectorSubcoreMesh(core_axis_name="c", subcore_axis_name="s")

    @functools.partial(
        pl.kernel,
        out_type=(
            jax.ShapeDtypeStruct((e_pad, D), jnp.float32),
            jax.ShapeDtypeStruct((e_pad, D), jnp.float32),
            jax.ShapeDtypeStruct((e_pad * 16,), jnp.float32),
        ),
        mesh=mesh,
        scratch_types=[
            pltpu.VMEM((ch, CHUNK), jnp.int32),
            pltpu.VMEM((ch, CHUNK), jnp.int32),
            pltpu.VMEM((CHUNK, D), jnp.float32),
            pltpu.VMEM((CHUNK, D), jnp.float32),
            pltpu.VMEM((CHUNK, D), jnp.float32),
            pltpu.VMEM((CHUNK, D), jnp.float32),
            pltpu.VMEM((n4,), jnp.float32),
            pltpu.VMEM((CHUNK * 16,), jnp.float32),
            pltpu.SemaphoreType.DMA,
            pltpu.SemaphoreType.DMA,
            pltpu.SemaphoreType.DMA,
            pltpu.SemaphoreType.DMA,
        ],
        compiler_params=_sc_compiler_params(),
    )
    def k(p1_hbm, p2_hbm, c4_hbm, ri_hbm, ci_hbm, g1_hbm, g2_hbm, cd_hbm,
          ir_v, ic_v, b1a, b2a, b1b, b2b, cl_v, cdb_v,
          sa1, sa2, sb1, sb2):
        wid = lax.axis_index("c") * NS + lax.axis_index("s")
        pltpu.sync_copy(ri_hbm.at[pl.ds(wid * ch, ch)], ir_v)
        pltpu.sync_copy(ci_hbm.at[pl.ds(wid * ch, ch)], ic_v)
        pltpu.sync_copy(c4_hbm, cl_v)

        zero16 = jnp.zeros((16,), jnp.float32)

        @pl.loop(0, CHUNK * 16, step=16)
        def _(i):
            cdb_v[pl.ds(i, 16)] = zero16

        iota16 = lax.iota(jnp.int32, 16)
        base = wid * ept

        def coord_math(jj):
            for sub in range(CHUNK // 16):
                er = ir_v[jj, pl.ds(sub * 16, 16)]
                ec = ic_v[jj, pl.ds(sub * 16, 16)]
                pos = sub * 256 + iota16 * 16
                for c in range(3):
                    xr = plsc.load_gather(cl_v, [er * 4 + c])
                    xc = plsc.load_gather(cl_v, [ec * 4 + c])
                    plsc.store_scatter(cdb_v, [pos + c], xr - xc)

        def issue(jj, buf1, buf2, s1, s2):
            pltpu.async_copy(p1_hbm.at[ir_v.at[jj]], buf1, s1)
            pltpu.async_copy(p2_hbm.at[ic_v.at[jj]], buf2, s2)

        def drain_and_write(jj, buf1, buf2, s1, s2):
            dst = pl.ds(base + jj * CHUNK, CHUNK)
            pltpu.make_async_copy(p1_hbm.at[ir_v.at[jj]], buf1, s1).wait()
            pltpu.make_async_copy(p2_hbm.at[ic_v.at[jj]], buf2, s2).wait()
            pltpu.sync_copy(buf1, g1_hbm.at[dst])
            pltpu.sync_copy(buf2, g2_hbm.at[dst])
            pltpu.sync_copy(
                cdb_v,
                cd_hbm.at[pl.ds((base + jj * CHUNK) * 16, CHUNK * 16)])

        issue(0, b1a, b2a, sa1, sa2)

        @pl.loop(0, ch, step=2)
        def _(j):
            issue(j + 1, b1b, b2b, sb1, sb2)
            coord_math(j)
            drain_and_write(j, b1a, b2a, sa1, sa2)

            @pl.when(j + 2 < ch)
            def _():
                issue(j + 2, b1a, b2a, sa1, sa2)

            coord_math(j + 1)
            drain_and_write(j + 1, b1b, b2b, sb1, sb2)

    return k(p1, p2, c4flat, rows2d, cols2d)


def _edge_mlp(g1, g2, cd, w1r, We2, be2r, Wc1, bc1r, wc2r, bc2r, n_edges,
              blk):
    e_pad, D = g1.shape
    grid = (e_pad // blk,)

    def body(g1_ref, g2_ref, cd_ref, w1_ref, w2_ref, b2_ref, wc1_ref,
             bc1_ref, wc2_ref, bc2_ref, ef_ref, tr_ref):
        s = g1_ref[...] + g2_ref[...]
        cdv = cd_ref[...]
        radial = jnp.sum(cdv * cdv, axis=1, keepdims=True)
        ef = _silu(s + radial * w1_ref[...])
        edge_feat = _silu(
            jnp.dot(ef, w2_ref[...], preferred_element_type=jnp.float32)
            + b2_ref[...])
        tt = _silu(
            jnp.dot(edge_feat, wc1_ref[...], preferred_element_type=jnp.float32)
            + bc1_ref[...])
        t = jnp.sum(tt * wc2_ref[...], axis=1, keepdims=True) + bc2_ref[...]
        scale = t / jnp.sqrt(radial + 1e-8)
        trans = cdv * scale
        eid = pl.program_id(0) * blk + lax.broadcasted_iota(
            jnp.int32, (blk, 1), 0)
        valid = eid < n_edges
        ef_ref[...] = jnp.where(valid, edge_feat, 0.0)
        tr_ref[...] = jnp.where(valid, trans, 0.0)

    const = pl.BlockSpec((1, D), lambda i: (0, 0))
    return pl.pallas_call(
        body,
        grid=grid,
        in_specs=[
            pl.BlockSpec((blk, D), lambda i: (i, 0)),
            pl.BlockSpec((blk, D), lambda i: (i, 0)),
            pl.BlockSpec((blk, 16), lambda i: (i, 0)),
            const,
            pl.BlockSpec((D, D), lambda i: (0, 0)),
            const,
            pl.BlockSpec((D, D), lambda i: (0, 0)),
            const,
            const,
            pl.BlockSpec((1, 1), lambda i: (0, 0)),
        ],
        out_specs=[
            pl.BlockSpec((blk, D), lambda i: (i, 0)),
            pl.BlockSpec((blk, 16), lambda i: (i, 0)),
        ],
        out_shape=(
            jax.ShapeDtypeStruct((e_pad, D), jnp.float32),
            jax.ShapeDtypeStruct((e_pad, 16), jnp.float32),
        ),
        compiler_params=pltpu.CompilerParams(
            dimension_semantics=("parallel",)),
    )(g1, g2, cd, w1r, We2, be2r, Wc1, bc1r, wc2r, bc2r)


def _sc_segment_sum_h(ef, rows2d, zh, npad):
    e_pad, D = ef.shape
    ept = e_pad // NW
    ch = ept // CHUNK
    npt = npad // NS
    mesh = plsc.VectorSubcoreMesh(core_axis_name="c", subcore_axis_name="s")

    @functools.partial(
        pl.kernel,
        out_type=jax.ShapeDtypeStruct((NC * npad, D), jnp.float32),
        mesh=mesh,
        scratch_types=[
            pltpu.VMEM_SHARED((npad, D), jnp.float32),
            pltpu.VMEM((ch, CHUNK), jnp.int32),
            pltpu.VMEM((CHUNK, D), jnp.float32),
            pltpu.VMEM((CHUNK, D), jnp.float32),
            pltpu.SemaphoreType.DMA,
            pltpu.SemaphoreType.DMA,
        ],
        compiler_params=_sc_compiler_params(),
    )
    def k(ef_hbm, ri_hbm, zh_hbm, ph_hbm, acc_h, idx_v, vha, vhb, sa, sb):
        cid = lax.axis_index("c")
        sid = lax.axis_index("s")
        wid = cid * NS + sid
        nslc = pl.ds(sid * npt, npt)
        pltpu.sync_copy(zh_hbm, acc_h.at[nslc])
        pltpu.sync_copy(ri_hbm.at[pl.ds(wid * ch, ch)], idx_v)
        plsc.subcore_barrier()
        base = wid * ept

        def src(jj):
            return ef_hbm.at[pl.ds(base + jj * CHUNK, CHUNK)]

        pltpu.async_copy(src(0), vha, sa)

        @pl.loop(0, ch, step=2)
        def _(j):
            pltpu.async_copy(src(j + 1), vhb, sb)
            pltpu.make_async_copy(src(j), vha, sa).wait()
            pltpu.sync_copy(vha, acc_h.at[idx_v.at[j]], add=True)

            @pl.when(j + 2 < ch)
            def _():
                pltpu.async_copy(src(j + 2), vha, sa)

            pltpu.make_async_copy(src(j + 1), vhb, sb).wait()
            pltpu.sync_copy(vhb, acc_h.at[idx_v.at[j + 1]], add=True)

        plsc.subcore_barrier()
        pltpu.sync_copy(acc_h.at[nslc],
                        ph_hbm.at[pl.ds(cid * npad + sid * npt, npt)])

    return k(ef, rows2d, zh)


def _sc_segment_sum_c(trflat, rows2d, zc, npad):
    e_pad16 = trflat.shape[0]
    e_pad = e_pad16 // 16
    ept = e_pad // NW
    ch = ept // CHUNK
    n4 = npad * 4
    mesh = plsc.VectorSubcoreMesh(core_axis_name="c", subcore_axis_name="s")

    @functools.partial(
        pl.kernel,
        out_type=jax.ShapeDtypeStruct((NW * n4,), jnp.float32),
        mesh=mesh,
        scratch_types=[
            pltpu.VMEM((n4,), jnp.float32),
            pltpu.VMEM((ch, CHUNK), jnp.int32),
            pltpu.VMEM((CHUNK * 16,), jnp.float32),
            pltpu.VMEM((CHUNK * 16,), jnp.float32),
            pltpu.SemaphoreType.DMA,
            pltpu.SemaphoreType.DMA,
        ],
        compiler_params=_sc_compiler_params(),
    )
    def k(tr_hbm, ri_hbm, zc_hbm, pc_hbm, acc_c, idx_v, vta, vtb, sa, sb):
        wid = lax.axis_index("c") * NS + lax.axis_index("s")
        pltpu.sync_copy(zc_hbm, acc_c)
        pltpu.sync_copy(ri_hbm.at[pl.ds(wid * ch, ch)], idx_v)
        iota16 = lax.iota(jnp.int32, 16)
        base = wid * ept

        def src(jj):
            return tr_hbm.at[pl.ds((base + jj * CHUNK) * 16, CHUNK * 16)]

        def accumulate(jj, vt_v):
            for sub in range(CHUNK // 16):
                en = idx_v[jj, pl.ds(sub * 16, 16)]
                pos = sub * 256 + iota16 * 16
                for c in range(3):
                    v = plsc.load_gather(vt_v, [pos + c])
                    plsc.addupdate_scatter(acc_c, [en * 4 + c], v)

        pltpu.async_copy(src(0), vta, sa)

        @pl.loop(0, ch, step=2)
        def _(j):
            pltpu.async_copy(src(j + 1), vtb, sb)
            pltpu.make_async_copy(src(j), vta, sa).wait()
            accumulate(j, vta)

            @pl.when(j + 2 < ch)
            def _():
                pltpu.async_copy(src(j + 2), vta, sa)

            pltpu.make_async_copy(src(j + 1), vtb, sb).wait()
            accumulate(j + 1, vtb)

        pltpu.sync_copy(acc_c, pc_hbm.at[pl.ds(wid * n4, n4)])

    return k(trflat, rows2d, zc)


def _node_mlp(h, c4mat, ph, pcmat, Wn1a, Wn1b, bn1r, Wn2, bn2r, npad):
    N, D = h.shape
    rows4 = c4mat.shape[0]

    def body(h_ref, c_ref, ph_ref, pc_ref, wa_ref, wb_ref, b1_ref,
             w2_ref, b2_ref, ho_ref, co_ref):
        hh = h_ref[...]
        agg = ph_ref[:N, :] + ph_ref[npad:npad + N, :]
        m1 = _silu(
            jnp.dot(hh, wa_ref[...], preferred_element_type=jnp.float32)
            + jnp.dot(agg, wb_ref[...], preferred_element_type=jnp.float32)
            + b1_ref[...])
        m = jnp.dot(m1, w2_ref[...], preferred_element_type=jnp.float32)
        ho_ref[...] = hh + m + b2_ref[...]
        co_ref[...] = c_ref[...] + jnp.sum(pc_ref[...], axis=0)

    return pl.pallas_call(
        body,
        out_shape=(
            jax.ShapeDtypeStruct((N, D), jnp.float32),
            jax.ShapeDtypeStruct((rows4, 128), jnp.float32),
        ),
    )(h, c4mat, ph, pcmat, Wn1a, Wn1b, bn1r, Wn2, bn2r)


def kernel(h, edge_index, coord, edge_attr,
           We1, be1, We2, be2, Wn1, bn1, Wn2, bn2, Wc1, bc1, Wc2, bc2):
    del edge_attr
    N, D = h.shape
    E = edge_index.shape[1]
    tile_edges = NW * CHUNK * 8
    e_pad = ((E + tile_edges - 1) // tile_edges) * tile_edges
    npad = ((N + NS * 8 - 1) // (NS * 8)) * (NS * 8)

    row = edge_index[0].astype(jnp.int32)
    col = edge_index[1].astype(jnp.int32)
    rows2d = jnp.pad(row, (0, e_pad - E)).reshape(e_pad // CHUNK, CHUNK)
    cols2d = jnp.pad(col, (0, e_pad - E)).reshape(e_pad // CHUNK, CHUNK)

    c4flat = jnp.pad(coord, ((0, npad - N), (0, 1))).reshape(-1)
    We1a = We1[:D]
    We1b = We1[D:2 * D]
    w1r = We1[2 * D].reshape(1, D)
    be1r = be1.reshape(1, D)
    be2r = be2.reshape(1, D)
    bc1r = bc1.reshape(1, D)
    wc2r = Wc2.reshape(1, D)
    bc2r = bc2.reshape(1, 1)
    bn1r = bn1.reshape(1, D)
    bn2r = bn2.reshape(1, D)
    Wn1a = Wn1[:D]
    Wn1b = Wn1[D:]

    p1, p2 = _build_tables(h, We1a, We1b, be1r)
    g1, g2, cdflat = _sc_gather(p1, p2, c4flat, rows2d, cols2d, e_pad)
    cd = cdflat.reshape(e_pad, 16)
    ef, tr = _edge_mlp(g1, g2, cd, w1r, We2, be2r, Wc1, bc1r, wc2r, bc2r,
                       E, 2048)
    zh = jnp.zeros((npad // NS, D), jnp.float32)
    zc = jnp.zeros((npad * 4,), jnp.float32)
    ph = _sc_segment_sum_h(ef, rows2d, zh, npad)
    pc = _sc_segment_sum_c(tr.reshape(-1), rows2d, zc, npad)
    rows4 = npad * 4 // 128
    pcmat = pc.reshape(NW, rows4, 128)
    c4mat = c4flat.reshape(rows4, 128)
    h_out, co_mat = _node_mlp(h, c4mat, ph, pcmat, Wn1a, Wn1b, bn1r,
                              Wn2, bn2r, npad)
    coord_out = co_mat.reshape(npad, 4)[:N, :3]
    return (h_out, coord_out)


# speedup vs baseline: 3.5764x; 1.0539x over previous
import dataclasses
import functools

import jax
import jax.numpy as jnp
from jax import lax
from jax.experimental import pallas as pl
from jax.experimental.pallas import tpu as pltpu
from jax.experimental.pallas import tpu_sc as plsc

NC = 2
NS = 16
NW = NC * NS
CHUNK = 128


def _sc_compiler_params():
    cp = pltpu.CompilerParams()
    if "needs_layout_passes" in pltpu.CompilerParams.__dataclass_fields__:
        cp = dataclasses.replace(cp, needs_layout_passes=False)
    return cp


def _silu(x):
    return x * jax.nn.sigmoid(x)


def _build_tables(h, We1a, We1b, be1r):
    N, D = h.shape

    def body(h_ref, wa_ref, wb_ref, b1_ref, p1_ref, p2_ref):
        hh = h_ref[...]
        p1_ref[...] = jnp.dot(hh, wa_ref[...],
                              preferred_element_type=jnp.float32)
        p2_ref[...] = jnp.dot(hh, wb_ref[...],
                              preferred_element_type=jnp.float32) + b1_ref[...]

    return pl.pallas_call(
        body,
        out_shape=(
            jax.ShapeDtypeStruct((N, D), jnp.float32),
            jax.ShapeDtypeStruct((N, D), jnp.float32),
        ),
    )(h, We1a, We1b, be1r)


def _sc_gather(p1, p2, rows2d, cols2d, e_pad):
    N, D = p1.shape
    ept = e_pad // NW
    ch = ept // CHUNK
    mesh = plsc.VectorSubcoreMesh(core_axis_name="c", subcore_axis_name="s")

    @functools.partial(
        pl.kernel,
        out_type=jax.ShapeDtypeStruct((e_pad, D), jnp.float32),
        mesh=mesh,
        scratch_types=[
            pltpu.VMEM_SHARED((NS * CHUNK, D), jnp.float32),
            pltpu.VMEM((ch, CHUNK), jnp.int32),
            pltpu.VMEM((ch, CHUNK), jnp.int32),
            pltpu.VMEM((CHUNK,), jnp.int32),
            pltpu.VMEM((CHUNK, D), jnp.float32),
            pltpu.VMEM((CHUNK, D), jnp.float32),
            pltpu.VMEM((CHUNK, D), jnp.float32),
            pltpu.VMEM((CHUNK, D), jnp.float32),
            pltpu.SemaphoreType.DMA,
            pltpu.SemaphoreType.DMA,
            pltpu.SemaphoreType.DMA,
            pltpu.SemaphoreType.DMA,
        ],
        compiler_params=_sc_compiler_params(),
    )
    def k(p1_hbm, p2_hbm, ri_hbm, ci_hbm, g_hbm,
          spm, ir_v, ic_v, id_v, b1a, b2a, b1b, b2b, sa1, sa2, sb1, sb2):
        cid = lax.axis_index("c")
        sid = lax.axis_index("s")
        wid = cid * NS + sid
        pltpu.sync_copy(ri_hbm.at[pl.ds(wid * ch, ch)], ir_v)
        pltpu.sync_copy(ci_hbm.at[pl.ds(wid * ch, ch)], ic_v)
        spm_t = spm.at[pl.ds(sid * CHUNK, CHUNK)]

        iota16 = lax.iota(jnp.int32, 16)

        @pl.loop(0, CHUNK, step=16)
        def _(i):
            id_v[pl.ds(i, 16)] = iota16 + i

        base = wid * ept

        def issue(jj, buf1, buf2, s1, s2):
            pltpu.async_copy(p1_hbm.at[ir_v.at[jj]], buf1, s1)
            pltpu.async_copy(p2_hbm.at[ic_v.at[jj]], buf2, s2)

        def finish(jj, buf1, buf2, s1, s2):
            pltpu.make_async_copy(p1_hbm.at[ir_v.at[jj]], buf1, s1).wait()
            pltpu.make_async_copy(p2_hbm.at[ic_v.at[jj]], buf2, s2).wait()
            pltpu.sync_copy(buf1, spm_t)
            pltpu.sync_copy(buf2, spm_t.at[id_v], add=True)
            pltpu.sync_copy(spm_t, g_hbm.at[pl.ds(base + jj * CHUNK, CHUNK)])

        issue(0, b1a, b2a, sa1, sa2)

        @pl.loop(0, ch, step=2)
        def _(j):
            issue(j + 1, b1b, b2b, sb1, sb2)
            finish(j, b1a, b2a, sa1, sa2)

            @pl.when(j + 2 < ch)
            def _():
                issue(j + 2, b1a, b2a, sa1, sa2)

            finish(j + 1, b1b, b2b, sb1, sb2)

    return k(p1, p2, rows2d, cols2d)


def _sc_coord_diff(c4flat, rows2d, cols2d, e_pad):
    n4 = c4flat.shape[0]
    ept = e_pad // NW
    ch = ept // CHUNK
    mesh = plsc.VectorSubcoreMesh(core_axis_name="c", subcore_axis_name="s")

    @functools.partial(
        pl.kernel,
        out_type=jax.ShapeDtypeStruct((e_pad * 16,), jnp.float32),
        mesh=mesh,
        scratch_types=[
            pltpu.VMEM((ch, CHUNK), jnp.int32),
            pltpu.VMEM((ch, CHUNK), jnp.int32),
            pltpu.VMEM((n4,), jnp.float32),
            pltpu.VMEM((CHUNK * 16,), jnp.float32),
            pltpu.VMEM((CHUNK * 16,), jnp.float32),
        ],
        compiler_params=_sc_compiler_params(),
    )
    def k(c4_hbm, ri_hbm, ci_hbm, cd_hbm, ir_v, ic_v, cl_v, cda, cdb):
        wid = lax.axis_index("c") * NS + lax.axis_index("s")
        pltpu.sync_copy(ri_hbm.at[pl.ds(wid * ch, ch)], ir_v)
        pltpu.sync_copy(ci_hbm.at[pl.ds(wid * ch, ch)], ic_v)
        pltpu.sync_copy(c4_hbm, cl_v)

        zero16 = jnp.zeros((16,), jnp.float32)

        @pl.loop(0, CHUNK * 16, step=16)
        def _(i):
            cda[pl.ds(i, 16)] = zero16
            cdb[pl.ds(i, 16)] = zero16

        iota16 = lax.iota(jnp.int32, 16)
        base = wid * ept

        def coord_math(jj, cdv):
            for sub in range(CHUNK // 16):
                er = ir_v[jj, pl.ds(sub * 16, 16)]
                ec = ic_v[jj, pl.ds(sub * 16, 16)]
                pos = sub * 256 + iota16 * 16
                for c in range(3):
                    xr = plsc.load_gather(cl_v, [er * 4 + c])
                    xc = plsc.load_gather(cl_v, [ec * 4 + c])
                    plsc.store_scatter(cdv, [pos + c], xr - xc)

        def write(jj, cdv):
            pltpu.sync_copy(
                cdv, cd_hbm.at[pl.ds((base + jj * CHUNK) * 16, CHUNK * 16)])

        @pl.loop(0, ch, step=2)
        def _(j):
            coord_math(j, cda)
            write(j, cda)
            coord_math(j + 1, cdb)
            write(j + 1, cdb)

    return k(c4flat, rows2d, cols2d)


def _edge_mlp(g, cd, w1r, We2, be2r, Wc1, bc1r, wc2r, bc2r, n_edges,
              blk):
    e_pad, D = g.shape
    grid = (e_pad // blk,)

    def body(g_ref, cd_ref, w1_ref, w2_ref, b2_ref, wc1_ref,
             bc1_ref, wc2_ref, bc2_ref, ef_ref, tr_ref):
        s = g_ref[...]
        cdv = cd_ref[...]
        radial = jnp.sum(cdv * cdv, axis=1, keepdims=True)
        ef = _silu(s + radial * w1_ref[...])
        edge_feat = _silu(
            jnp.dot(ef.astype(jnp.bfloat16),
                    w2_ref[...].astype(jnp.bfloat16),
                    preferred_element_type=jnp.float32)
            + b2_ref[...])
        tt = _silu(
            jnp.dot(edge_feat.astype(jnp.bfloat16),
                    wc1_ref[...].astype(jnp.bfloat16),
                    preferred_element_type=jnp.float32)
            + bc1_ref[...])
        t = jnp.sum(tt * wc2_ref[...], axis=1, keepdims=True) + bc2_ref[...]
        scale = t / jnp.sqrt(radial + 1e-8)
        trans = cdv * scale
        eid = pl.program_id(0) * blk + lax.broadcasted_iota(
            jnp.int32, (blk, 1), 0)
        valid = eid < n_edges
        ef_ref[...] = jnp.where(valid, edge_feat, 0.0)
        tr_ref[...] = jnp.where(valid, trans, 0.0)

    const = pl.BlockSpec((1, D), lambda i: (0, 0))
    return pl.pallas_call(
        body,
        grid=grid,
        in_specs=[
            pl.BlockSpec((blk, D), lambda i: (i, 0)),
            pl.BlockSpec((blk, 16), lambda i: (i, 0)),
            const,
            pl.BlockSpec((D, D), lambda i: (0, 0)),
            const,
            pl.BlockSpec((D, D), lambda i: (0, 0)),
            const,
            const,
            pl.BlockSpec((1, 1), lambda i: (0, 0)),
        ],
        out_specs=[
            pl.BlockSpec((blk, D), lambda i: (i, 0)),
            pl.BlockSpec((blk, 16), lambda i: (i, 0)),
        ],
        out_shape=(
            jax.ShapeDtypeStruct((e_pad, D), jnp.float32),
            jax.ShapeDtypeStruct((e_pad, 16), jnp.float32),
        ),
        compiler_params=pltpu.CompilerParams(
            dimension_semantics=("parallel",)),
    )(g, cd, w1r, We2, be2r, Wc1, bc1r, wc2r, bc2r)


def _sc_segment_sum_h(ef, rows2d, zh, npad):
    e_pad, D = ef.shape
    ept = e_pad // NW
    ch = ept // CHUNK
    npt = npad // NS
    mesh = plsc.VectorSubcoreMesh(core_axis_name="c", subcore_axis_name="s")

    @functools.partial(
        pl.kernel,
        out_type=jax.ShapeDtypeStruct((NC * npad, D), jnp.float32),
        mesh=mesh,
        scratch_types=[
            pltpu.VMEM_SHARED((npad, D), jnp.float32),
            pltpu.VMEM((ch, CHUNK), jnp.int32),
            pltpu.VMEM((CHUNK, D), jnp.float32),
            pltpu.VMEM((CHUNK, D), jnp.float32),
            pltpu.SemaphoreType.DMA,
            pltpu.SemaphoreType.DMA,
        ],
        compiler_params=_sc_compiler_params(),
    )
    def k(ef_hbm, ri_hbm, zh_hbm, ph_hbm, acc_h, idx_v, vha, vhb, sa, sb):
        cid = lax.axis_index("c")
        sid = lax.axis_index("s")
        wid = cid * NS + sid
        nslc = pl.ds(sid * npt, npt)
        pltpu.sync_copy(zh_hbm, acc_h.at[nslc])
        pltpu.sync_copy(ri_hbm.at[pl.ds(wid * ch, ch)], idx_v)
        plsc.subcore_barrier()
        base = wid * ept

        def src(jj):
            return ef_hbm.at[pl.ds(base + jj * CHUNK, CHUNK)]

        pltpu.async_copy(src(0), vha, sa)

        @pl.loop(0, ch, step=2)
        def _(j):
            pltpu.async_copy(src(j + 1), vhb, sb)
            pltpu.make_async_copy(src(j), vha, sa).wait()
            pltpu.sync_copy(vha, acc_h.at[idx_v.at[j]], add=True)

            @pl.when(j + 2 < ch)
            def _():
                pltpu.async_copy(src(j + 2), vha, sa)

            pltpu.make_async_copy(src(j + 1), vhb, sb).wait()
            pltpu.sync_copy(vhb, acc_h.at[idx_v.at[j + 1]], add=True)

        plsc.subcore_barrier()
        pltpu.sync_copy(acc_h.at[nslc],
                        ph_hbm.at[pl.ds(cid * npad + sid * npt, npt)])

    return k(ef, rows2d, zh)


def _sc_segment_sum_c(trflat, rows2d, zc, npad):
    e_pad16 = trflat.shape[0]
    e_pad = e_pad16 // 16
    ept = e_pad // NW
    ch = ept // CHUNK
    n4 = npad * 4
    mesh = plsc.VectorSubcoreMesh(core_axis_name="c", subcore_axis_name="s")

    @functools.partial(
        pl.kernel,
        out_type=jax.ShapeDtypeStruct((NW * n4,), jnp.float32),
        mesh=mesh,
        scratch_types=[
            pltpu.VMEM((n4,), jnp.float32),
            pltpu.VMEM((ch, CHUNK), jnp.int32),
            pltpu.VMEM((CHUNK * 16,), jnp.float32),
            pltpu.VMEM((CHUNK * 16,), jnp.float32),
            pltpu.SemaphoreType.DMA,
            pltpu.SemaphoreType.DMA,
        ],
        compiler_params=_sc_compiler_params(),
    )
    def k(tr_hbm, ri_hbm, zc_hbm, pc_hbm, acc_c, idx_v, vta, vtb, sa, sb):
        wid = lax.axis_index("c") * NS + lax.axis_index("s")
        pltpu.sync_copy(zc_hbm, acc_c)
        pltpu.sync_copy(ri_hbm.at[pl.ds(wid * ch, ch)], idx_v)
        iota16 = lax.iota(jnp.int32, 16)
        base = wid * ept

        def src(jj):
            return tr_hbm.at[pl.ds((base + jj * CHUNK) * 16, CHUNK * 16)]

        def accumulate(jj, vt_v):
            for sub in range(CHUNK // 16):
                en = idx_v[jj, pl.ds(sub * 16, 16)]
                pos = sub * 256 + iota16 * 16
                for c in range(3):
                    v = plsc.load_gather(vt_v, [pos + c])
                    plsc.addupdate_scatter(acc_c, [en * 4 + c], v)

        pltpu.async_copy(src(0), vta, sa)

        @pl.loop(0, ch, step=2)
        def _(j):
            pltpu.async_copy(src(j + 1), vtb, sb)
            pltpu.make_async_copy(src(j), vta, sa).wait()
            accumulate(j, vta)

            @pl.when(j + 2 < ch)
            def _():
                pltpu.async_copy(src(j + 2), vta, sa)

            pltpu.make_async_copy(src(j + 1), vtb, sb).wait()
            accumulate(j + 1, vtb)

        pltpu.sync_copy(acc_c, pc_hbm.at[pl.ds(wid * n4, n4)])

    return k(trflat, rows2d, zc)


def _node_mlp(h, c4mat, ph, pcmat, Wn1a, Wn1b, bn1r, Wn2, bn2r, npad):
    N, D = h.shape
    rows4 = c4mat.shape[0]

    def body(h_ref, c_ref, ph_ref, pc_ref, wa_ref, wb_ref, b1_ref,
             w2_ref, b2_ref, ho_ref, co_ref):
        hh = h_ref[...]
        agg = ph_ref[:N, :] + ph_ref[npad:npad + N, :]
        m1 = _silu(
            jnp.dot(hh, wa_ref[...], preferred_element_type=jnp.float32)
            + jnp.dot(agg, wb_ref[...], preferred_element_type=jnp.float32)
            + b1_ref[...])
        m = jnp.dot(m1, w2_ref[...], preferred_element_type=jnp.float32)
        ho_ref[...] = hh + m + b2_ref[...]
        co_ref[...] = c_ref[...] + jnp.sum(pc_ref[...], axis=0)

    return pl.pallas_call(
        body,
        out_shape=(
            jax.ShapeDtypeStruct((N, D), jnp.float32),
            jax.ShapeDtypeStruct((rows4, 128), jnp.float32),
        ),
    )(h, c4mat, ph, pcmat, Wn1a, Wn1b, bn1r, Wn2, bn2r)


def kernel(h, edge_index, coord, edge_attr,
           We1, be1, We2, be2, Wn1, bn1, Wn2, bn2, Wc1, bc1, Wc2, bc2):
    del edge_attr
    N, D = h.shape
    E = edge_index.shape[1]
    tile_edges = NW * CHUNK * 8
    e_pad = ((E + tile_edges - 1) // tile_edges) * tile_edges
    npad = ((N + NS * 8 - 1) // (NS * 8)) * (NS * 8)

    row = edge_index[0].astype(jnp.int32)
    col = edge_index[1].astype(jnp.int32)
    rows2d = jnp.pad(row, (0, e_pad - E)).reshape(e_pad // CHUNK, CHUNK)
    cols2d = jnp.pad(col, (0, e_pad - E)).reshape(e_pad // CHUNK, CHUNK)

    c4flat = jnp.pad(coord, ((0, npad - N), (0, 1))).reshape(-1)
    We1a = We1[:D]
    We1b = We1[D:2 * D]
    w1r = We1[2 * D].reshape(1, D)
    be1r = be1.reshape(1, D)
    be2r = be2.reshape(1, D)
    bc1r = bc1.reshape(1, D)
    wc2r = Wc2.reshape(1, D)
    bc2r = bc2.reshape(1, 1)
    bn1r = bn1.reshape(1, D)
    bn2r = bn2.reshape(1, D)
    Wn1a = Wn1[:D]
    Wn1b = Wn1[D:]

    p1, p2 = _build_tables(h, We1a, We1b, be1r)
    g = _sc_gather(p1, p2, rows2d, cols2d, e_pad)
    cdflat = _sc_coord_diff(c4flat, rows2d, cols2d, e_pad)
    cd = cdflat.reshape(e_pad, 16)
    ef, tr = _edge_mlp(g, cd, w1r, We2, be2r, Wc1, bc1r, wc2r, bc2r,
                       E, 2048)
    zh = jnp.zeros((npad // NS, D), jnp.float32)
    zc = jnp.zeros((npad * 4,), jnp.float32)
    ph = _sc_segment_sum_h(ef, rows2d, zh, npad)
    pc = _sc_segment_sum_c(tr.reshape(-1), rows2d, zc, npad)
    rows4 = npad * 4 // 128
    pcmat = pc.reshape(NW, rows4, 128)
    c4mat = c4flat.reshape(rows4, 128)
    h_out, co_mat = _node_mlp(h, c4mat, ph, pcmat, Wn1a, Wn1b, bn1r,
                              Wn2, bn2r, npad)
    coord_out = co_mat.reshape(npad, 4)[:N, :3]
    return (h_out, coord_out)


# speedup vs baseline: 4.2700x; 1.1940x over previous
import dataclasses
import functools

import jax
import jax.numpy as jnp
from jax import lax
from jax.experimental import pallas as pl
from jax.experimental.pallas import tpu as pltpu
from jax.experimental.pallas import tpu_sc as plsc

NC = 2
NS = 16
NW = NC * NS
CHUNK = 128


def _sc_compiler_params():
    cp = pltpu.CompilerParams()
    if "needs_layout_passes" in pltpu.CompilerParams.__dataclass_fields__:
        cp = dataclasses.replace(cp, needs_layout_passes=False)
    return cp


def _silu(x):
    return x * jax.nn.sigmoid(x)


def _build_tables(h, We1a, We1b, be1r, npad):
    N, D = h.shape

    def body(h_ref, wa_ref, wb_ref, b1_ref, tab_ref):
        hh = h_ref[...]
        p1 = jnp.dot(hh, wa_ref[...], preferred_element_type=jnp.float32)
        p2 = jnp.dot(hh, wb_ref[...],
                     preferred_element_type=jnp.float32) + b1_ref[...]
        z = jnp.zeros((npad - N, D), jnp.float32)
        tab_ref[...] = jnp.concatenate([p1, z, p2, z], axis=0)

    return pl.pallas_call(
        body,
        out_shape=jax.ShapeDtypeStruct((2 * npad, D), jnp.float32),
    )(h, We1a, We1b, be1r)


def _sc_gather(tab, idx2d, e_pad, npad):
    D = tab.shape[1]
    ept2 = 2 * e_pad // NW
    ch2 = ept2 // CHUNK
    chh = ch2 // 2
    npt = npad // NS
    mesh = plsc.VectorSubcoreMesh(core_axis_name="c", subcore_axis_name="s")

    @functools.partial(
        pl.kernel,
        out_type=jax.ShapeDtypeStruct((2 * e_pad, D), jnp.float32),
        mesh=mesh,
        scratch_types=[
            pltpu.VMEM_SHARED((npad, D), jnp.float32),
            pltpu.VMEM((chh, CHUNK), jnp.int32),
            pltpu.VMEM((CHUNK, D), jnp.float32),
            pltpu.VMEM((CHUNK, D), jnp.float32),
            pltpu.SemaphoreType.DMA,
            pltpu.SemaphoreType.DMA,
        ],
        compiler_params=_sc_compiler_params(),
    )
    def k(tab_hbm, ix_hbm, g_hbm, spm, ix_v, ba, bb, sa, sb):
        cid = lax.axis_index("c")
        sid = lax.axis_index("s")
        wid = cid * NS + sid
        pltpu.sync_copy(tab_hbm.at[pl.ds(cid * npad + sid * npt, npt)],
                        spm.at[pl.ds(sid * npt, npt)])
        plsc.subcore_barrier()
        base = wid * ept2

        def issue(jj, buf, s):
            pltpu.async_copy(spm.at[ix_v.at[jj]], buf, s)

        def finish(jj, half, buf, s):
            pltpu.make_async_copy(spm.at[ix_v.at[jj]], buf, s).wait()
            pltpu.sync_copy(
                buf,
                g_hbm.at[pl.ds(base + (half * chh + jj) * CHUNK, CHUNK)])

        for half in range(2):
            pltpu.sync_copy(
                ix_hbm.at[pl.ds(wid * ch2 + half * chh, chh)], ix_v)
            issue(0, ba, sa)

            @pl.loop(0, chh, step=2)
            def _(j):
                issue(j + 1, bb, sb)
                finish(j, half, ba, sa)

                @pl.when(j + 2 < chh)
                def _():
                    issue(j + 2, ba, sa)

                finish(j + 1, half, bb, sb)

    return k(tab, idx2d)


def _sc_coord_diff(c4flat, rows2d, cols2d, e_pad):
    n4 = c4flat.shape[0]
    ept = e_pad // NW
    ch = ept // CHUNK
    mesh = plsc.VectorSubcoreMesh(core_axis_name="c", subcore_axis_name="s")

    @functools.partial(
        pl.kernel,
        out_type=jax.ShapeDtypeStruct((e_pad * 16,), jnp.float32),
        mesh=mesh,
        scratch_types=[
            pltpu.VMEM((ch, CHUNK), jnp.int32),
            pltpu.VMEM((ch, CHUNK), jnp.int32),
            pltpu.VMEM((n4,), jnp.float32),
            pltpu.VMEM((CHUNK * 16,), jnp.float32),
            pltpu.VMEM((CHUNK * 16,), jnp.float32),
        ],
        compiler_params=_sc_compiler_params(),
    )
    def k(c4_hbm, ri_hbm, ci_hbm, cd_hbm, ir_v, ic_v, cl_v, cda, cdb):
        wid = lax.axis_index("c") * NS + lax.axis_index("s")
        pltpu.sync_copy(ri_hbm.at[pl.ds(wid * ch, ch)], ir_v)
        pltpu.sync_copy(ci_hbm.at[pl.ds(wid * ch, ch)], ic_v)
        pltpu.sync_copy(c4_hbm, cl_v)

        zero16 = jnp.zeros((16,), jnp.float32)

        @pl.loop(0, CHUNK * 16, step=16)
        def _(i):
            cda[pl.ds(i, 16)] = zero16
            cdb[pl.ds(i, 16)] = zero16

        iota16 = lax.iota(jnp.int32, 16)
        base = wid * ept

        def coord_math(jj, cdv):
            for sub in range(CHUNK // 16):
                er = ir_v[jj, pl.ds(sub * 16, 16)]
                ec = ic_v[jj, pl.ds(sub * 16, 16)]
                pos = sub * 256 + iota16 * 16
                for c in range(3):
                    xr = plsc.load_gather(cl_v, [er * 4 + c])
                    xc = plsc.load_gather(cl_v, [ec * 4 + c])
                    plsc.store_scatter(cdv, [pos + c], xr - xc)

        def write(jj, cdv):
            pltpu.sync_copy(
                cdv, cd_hbm.at[pl.ds((base + jj * CHUNK) * 16, CHUNK * 16)])

        @pl.loop(0, ch, step=2)
        def _(j):
            coord_math(j, cda)
            write(j, cda)
            coord_math(j + 1, cdb)
            write(j + 1, cdb)

    return k(c4flat, rows2d, cols2d)


def _edge_mlp(g1, g2, cd, w1r, We2, be2r, Wc1, bc1r, wc2r, bc2r, n_edges,
              blk):
    e_pad, D = g1.shape
    grid = (e_pad // blk,)

    def body(g1_ref, g2_ref, cd_ref, w1_ref, w2_ref, b2_ref, wc1_ref,
             bc1_ref, wc2_ref, bc2_ref, ef_ref, tr_ref):
        s = g1_ref[...] + g2_ref[...]
        cdv = cd_ref[...]
        radial = jnp.sum(cdv * cdv, axis=1, keepdims=True)
        ef = _silu(s + radial * w1_ref[...])
        edge_feat = _silu(
            jnp.dot(ef.astype(jnp.bfloat16),
                    w2_ref[...].astype(jnp.bfloat16),
                    preferred_element_type=jnp.float32)
            + b2_ref[...])
        tt = _silu(
            jnp.dot(edge_feat.astype(jnp.bfloat16),
                    wc1_ref[...].astype(jnp.bfloat16),
                    preferred_element_type=jnp.float32)
            + bc1_ref[...])
        t = jnp.sum(tt * wc2_ref[...], axis=1, keepdims=True) + bc2_ref[...]
        scale = t / jnp.sqrt(radial + 1e-8)
        trans = cdv * scale
        eid = pl.program_id(0) * blk + lax.broadcasted_iota(
            jnp.int32, (blk, 1), 0)
        valid = eid < n_edges
        ef_ref[...] = jnp.where(valid, edge_feat, 0.0)
        tr_ref[...] = jnp.where(valid, trans, 0.0)

    const = pl.BlockSpec((1, D), lambda i: (0, 0))
    return pl.pallas_call(
        body,
        grid=grid,
        in_specs=[
            pl.BlockSpec((blk, D), lambda i: (i, 0)),
            pl.BlockSpec((blk, D), lambda i: (i, 0)),
            pl.BlockSpec((blk, 16), lambda i: (i, 0)),
            const,
            pl.BlockSpec((D, D), lambda i: (0, 0)),
            const,
            pl.BlockSpec((D, D), lambda i: (0, 0)),
            const,
            const,
            pl.BlockSpec((1, 1), lambda i: (0, 0)),
        ],
        out_specs=[
            pl.BlockSpec((blk, D), lambda i: (i, 0)),
            pl.BlockSpec((blk, 16), lambda i: (i, 0)),
        ],
        out_shape=(
            jax.ShapeDtypeStruct((e_pad, D), jnp.float32),
            jax.ShapeDtypeStruct((e_pad, 16), jnp.float32),
        ),
        compiler_params=pltpu.CompilerParams(
            dimension_semantics=("parallel",)),
    )(g1, g2, cd, w1r, We2, be2r, Wc1, bc1r, wc2r, bc2r)


def _sc_segment_sum_h(ef, rows2d, zh, npad):
    e_pad, D = ef.shape
    ept = e_pad // NW
    ch = ept // CHUNK
    npt = npad // NS
    mesh = plsc.VectorSubcoreMesh(core_axis_name="c", subcore_axis_name="s")

    @functools.partial(
        pl.kernel,
        out_type=jax.ShapeDtypeStruct((NC * npad, D), jnp.float32),
        mesh=mesh,
        scratch_types=[
            pltpu.VMEM_SHARED((npad, D), jnp.float32),
            pltpu.VMEM((ch, CHUNK), jnp.int32),
            pltpu.VMEM((CHUNK, D), jnp.float32),
            pltpu.VMEM((CHUNK, D), jnp.float32),
            pltpu.SemaphoreType.DMA,
            pltpu.SemaphoreType.DMA,
        ],
        compiler_params=_sc_compiler_params(),
    )
    def k(ef_hbm, ri_hbm, zh_hbm, ph_hbm, acc_h, idx_v, vha, vhb, sa, sb):
        cid = lax.axis_index("c")
        sid = lax.axis_index("s")
        wid = cid * NS + sid
        nslc = pl.ds(sid * npt, npt)
        pltpu.sync_copy(zh_hbm, acc_h.at[nslc])
        pltpu.sync_copy(ri_hbm.at[pl.ds(wid * ch, ch)], idx_v)
        plsc.subcore_barrier()
        base = wid * ept

        def src(jj):
            return ef_hbm.at[pl.ds(base + jj * CHUNK, CHUNK)]

        pltpu.async_copy(src(0), vha, sa)

        @pl.loop(0, ch, step=2)
        def _(j):
            pltpu.async_copy(src(j + 1), vhb, sb)
            pltpu.make_async_copy(src(j), vha, sa).wait()
            pltpu.sync_copy(vha, acc_h.at[idx_v.at[j]], add=True)

            @pl.when(j + 2 < ch)
            def _():
                pltpu.async_copy(src(j + 2), vha, sa)

            pltpu.make_async_copy(src(j + 1), vhb, sb).wait()
            pltpu.sync_copy(vhb, acc_h.at[idx_v.at[j + 1]], add=True)

        plsc.subcore_barrier()
        pltpu.sync_copy(acc_h.at[nslc],
                        ph_hbm.at[pl.ds(cid * npad + sid * npt, npt)])

    return k(ef, rows2d, zh)


def _sc_segment_sum_c(trflat, rows2d, zc, npad):
    e_pad16 = trflat.shape[0]
    e_pad = e_pad16 // 16
    ept = e_pad // NW
    ch = ept // CHUNK
    n4 = npad * 4
    mesh = plsc.VectorSubcoreMesh(core_axis_name="c", subcore_axis_name="s")

    @functools.partial(
        pl.kernel,
        out_type=jax.ShapeDtypeStruct((NW * n4,), jnp.float32),
        mesh=mesh,
        scratch_types=[
            pltpu.VMEM((n4,), jnp.float32),
            pltpu.VMEM((ch, CHUNK), jnp.int32),
            pltpu.VMEM((CHUNK * 16,), jnp.float32),
            pltpu.VMEM((CHUNK * 16,), jnp.float32),
            pltpu.SemaphoreType.DMA,
            pltpu.SemaphoreType.DMA,
        ],
        compiler_params=_sc_compiler_params(),
    )
    def k(tr_hbm, ri_hbm, zc_hbm, pc_hbm, acc_c, idx_v, vta, vtb, sa, sb):
        wid = lax.axis_index("c") * NS + lax.axis_index("s")
        pltpu.sync_copy(zc_hbm, acc_c)
        pltpu.sync_copy(ri_hbm.at[pl.ds(wid * ch, ch)], idx_v)
        iota16 = lax.iota(jnp.int32, 16)
        base = wid * ept

        def src(jj):
            return tr_hbm.at[pl.ds((base + jj * CHUNK) * 16, CHUNK * 16)]

        def accumulate(jj, vt_v):
            for sub in range(CHUNK // 16):
                en = idx_v[jj, pl.ds(sub * 16, 16)]
                pos = sub * 256 + iota16 * 16
                for c in range(3):
                    v = plsc.load_gather(vt_v, [pos + c])
                    plsc.addupdate_scatter(acc_c, [en * 4 + c], v)

        pltpu.async_copy(src(0), vta, sa)

        @pl.loop(0, ch, step=2)
        def _(j):
            pltpu.async_copy(src(j + 1), vtb, sb)
            pltpu.make_async_copy(src(j), vta, sa).wait()
            accumulate(j, vta)

            @pl.when(j + 2 < ch)
            def _():
                pltpu.async_copy(src(j + 2), vta, sa)

            pltpu.make_async_copy(src(j + 1), vtb, sb).wait()
            accumulate(j + 1, vtb)

        pltpu.sync_copy(acc_c, pc_hbm.at[pl.ds(wid * n4, n4)])

    return k(trflat, rows2d, zc)


def _node_mlp(h, c4mat, ph, pcmat, Wn1a, Wn1b, bn1r, Wn2, bn2r, npad):
    N, D = h.shape
    rows4 = c4mat.shape[0]

    def body(h_ref, c_ref, ph_ref, pc_ref, wa_ref, wb_ref, b1_ref,
             w2_ref, b2_ref, ho_ref, co_ref):
        hh = h_ref[...]
        agg = ph_ref[:N, :] + ph_ref[npad:npad + N, :]
        m1 = _silu(
            jnp.dot(hh, wa_ref[...], preferred_element_type=jnp.float32)
            + jnp.dot(agg, wb_ref[...], preferred_element_type=jnp.float32)
            + b1_ref[...])
        m = jnp.dot(m1, w2_ref[...], preferred_element_type=jnp.float32)
        ho_ref[...] = hh + m + b2_ref[...]
        co_ref[...] = c_ref[...] + jnp.sum(pc_ref[...], axis=0)

    return pl.pallas_call(
        body,
        out_shape=(
            jax.ShapeDtypeStruct((N, D), jnp.float32),
            jax.ShapeDtypeStruct((rows4, 128), jnp.float32),
        ),
    )(h, c4mat, ph, pcmat, Wn1a, Wn1b, bn1r, Wn2, bn2r)


def kernel(h, edge_index, coord, edge_attr,
           We1, be1, We2, be2, Wn1, bn1, Wn2, bn2, Wc1, bc1, Wc2, bc2):
    del edge_attr
    N, D = h.shape
    E = edge_index.shape[1]
    tile_edges = NW * CHUNK * 8
    e_pad = ((E + tile_edges - 1) // tile_edges) * tile_edges
    npad = ((N + NS * 8 - 1) // (NS * 8)) * (NS * 8)

    row = edge_index[0].astype(jnp.int32)
    col = edge_index[1].astype(jnp.int32)
    rows2d = jnp.pad(row, (0, e_pad - E)).reshape(e_pad // CHUNK, CHUNK)
    cols2d = jnp.pad(col, (0, e_pad - E)).reshape(e_pad // CHUNK, CHUNK)

    c4flat = jnp.pad(coord, ((0, npad - N), (0, 1))).reshape(-1)
    We1a = We1[:D]
    We1b = We1[D:2 * D]
    w1r = We1[2 * D].reshape(1, D)
    be1r = be1.reshape(1, D)
    be2r = be2.reshape(1, D)
    bc1r = bc1.reshape(1, D)
    wc2r = Wc2.reshape(1, D)
    bc2r = bc2.reshape(1, 1)
    bn1r = bn1.reshape(1, D)
    bn2r = bn2.reshape(1, D)
    Wn1a = Wn1[:D]
    Wn1b = Wn1[D:]

    tab = _build_tables(h, We1a, We1b, be1r, npad)
    idx2d = jnp.concatenate([rows2d, cols2d], axis=0)
    g2x = _sc_gather(tab, idx2d, e_pad, npad)
    g1 = g2x[:e_pad]
    g2 = g2x[e_pad:]
    cdflat = _sc_coord_diff(c4flat, rows2d, cols2d, e_pad)
    cd = cdflat.reshape(e_pad, 16)
    ef, tr = _edge_mlp(g1, g2, cd, w1r, We2, be2r, Wc1, bc1r, wc2r, bc2r,
                       E, 2048)
    zh = jnp.zeros((npad // NS, D), jnp.float32)
    zc = jnp.zeros((npad * 4,), jnp.float32)
    ph = _sc_segment_sum_h(ef, rows2d, zh, npad)
    pc = _sc_segment_sum_c(tr.reshape(-1), rows2d, zc, npad)
    rows4 = npad * 4 // 128
    pcmat = pc.reshape(NW, rows4, 128)
    c4mat = c4flat.reshape(rows4, 128)
    h_out, co_mat = _node_mlp(h, c4mat, ph, pcmat, Wn1a, Wn1b, bn1r,
                              Wn2, bn2r, npad)
    coord_out = co_mat.reshape(npad, 4)[:N, :3]
    return (h_out, coord_out)


# speedup vs baseline: 4.2824x; 1.0029x over previous
import dataclasses
import functools

import jax
import jax.numpy as jnp
from jax import lax
from jax.experimental import pallas as pl
from jax.experimental.pallas import tpu as pltpu
from jax.experimental.pallas import tpu_sc as plsc

NC = 2
NS = 16
NW = NC * NS
CHUNK = 128


def _sc_compiler_params():
    cp = pltpu.CompilerParams()
    if "needs_layout_passes" in pltpu.CompilerParams.__dataclass_fields__:
        cp = dataclasses.replace(cp, needs_layout_passes=False)
    return cp


def _silu(x):
    return x * jax.nn.sigmoid(x)


def _build_tables(h, We1a, We1b, be1r, npad):
    N, D = h.shape

    def body(h_ref, wa_ref, wb_ref, b1_ref, tab_ref):
        hh = h_ref[...]
        p1 = jnp.dot(hh, wa_ref[...], preferred_element_type=jnp.float32)
        p2 = jnp.dot(hh, wb_ref[...],
                     preferred_element_type=jnp.float32) + b1_ref[...]
        z = jnp.zeros((npad - N, D), jnp.float32)
        tab_ref[...] = jnp.concatenate([p1, z, p2, z], axis=0)

    return pl.pallas_call(
        body,
        out_shape=jax.ShapeDtypeStruct((2 * npad, D), jnp.float32),
    )(h, We1a, We1b, be1r)


def _sc_gather(tab, idx2d, e_pad, npad):
    D = tab.shape[1]
    ept2 = 2 * e_pad // NW
    ch2 = ept2 // CHUNK
    chh = ch2 // 2
    npt = npad // NS
    mesh = plsc.VectorSubcoreMesh(core_axis_name="c", subcore_axis_name="s")

    @functools.partial(
        pl.kernel,
        out_type=jax.ShapeDtypeStruct((2 * e_pad, D), jnp.float32),
        mesh=mesh,
        scratch_types=[
            pltpu.VMEM_SHARED((npad, D), jnp.float32),
            pltpu.VMEM((chh, CHUNK), jnp.int32),
            pltpu.VMEM((CHUNK, D), jnp.float32),
            pltpu.VMEM((CHUNK, D), jnp.float32),
            pltpu.SemaphoreType.DMA,
            pltpu.SemaphoreType.DMA,
        ],
        compiler_params=_sc_compiler_params(),
    )
    def k(tab_hbm, ix_hbm, g_hbm, spm, ix_v, ba, bb, sa, sb):
        cid = lax.axis_index("c")
        sid = lax.axis_index("s")
        wid = cid * NS + sid
        pltpu.sync_copy(tab_hbm.at[pl.ds(cid * npad + sid * npt, npt)],
                        spm.at[pl.ds(sid * npt, npt)])
        plsc.subcore_barrier()
        base = wid * ept2

        def issue(jj, buf, s):
            pltpu.async_copy(spm.at[ix_v.at[jj]], buf, s)

        def finish(jj, half, buf, s):
            pltpu.make_async_copy(spm.at[ix_v.at[jj]], buf, s).wait()
            pltpu.sync_copy(
                buf,
                g_hbm.at[pl.ds(base + (half * chh + jj) * CHUNK, CHUNK)])

        for half in range(2):
            pltpu.sync_copy(
                ix_hbm.at[pl.ds(wid * ch2 + half * chh, chh)], ix_v)
            issue(0, ba, sa)

            @pl.loop(0, chh, step=2)
            def _(j):
                issue(j + 1, bb, sb)
                finish(j, half, ba, sa)

                @pl.when(j + 2 < chh)
                def _():
                    issue(j + 2, ba, sa)

                finish(j + 1, half, bb, sb)

    return k(tab, idx2d)


def _sc_coord_diff(c4flat, rows2d, cols2d, e_pad):
    n4 = c4flat.shape[0]
    ept = e_pad // NW
    ch = ept // CHUNK
    mesh = plsc.VectorSubcoreMesh(core_axis_name="c", subcore_axis_name="s")

    @functools.partial(
        pl.kernel,
        out_type=jax.ShapeDtypeStruct((e_pad * 16,), jnp.float32),
        mesh=mesh,
        scratch_types=[
            pltpu.VMEM((ch, CHUNK), jnp.int32),
            pltpu.VMEM((ch, CHUNK), jnp.int32),
            pltpu.VMEM((n4,), jnp.float32),
            pltpu.VMEM((CHUNK * 16,), jnp.float32),
            pltpu.VMEM((CHUNK * 16,), jnp.float32),
        ],
        compiler_params=_sc_compiler_params(),
    )
    def k(c4_hbm, ri_hbm, ci_hbm, cd_hbm, ir_v, ic_v, cl_v, cda, cdb):
        wid = lax.axis_index("c") * NS + lax.axis_index("s")
        pltpu.sync_copy(ri_hbm.at[pl.ds(wid * ch, ch)], ir_v)
        pltpu.sync_copy(ci_hbm.at[pl.ds(wid * ch, ch)], ic_v)
        pltpu.sync_copy(c4_hbm, cl_v)

        zero16 = jnp.zeros((16,), jnp.float32)

        @pl.loop(0, CHUNK * 16, step=16)
        def _(i):
            cda[pl.ds(i, 16)] = zero16
            cdb[pl.ds(i, 16)] = zero16

        iota16 = lax.iota(jnp.int32, 16)
        base = wid * ept

        def coord_math(jj, cdv):
            for sub in range(CHUNK // 16):
                er = ir_v[jj, pl.ds(sub * 16, 16)]
                ec = ic_v[jj, pl.ds(sub * 16, 16)]
                pos = sub * 256 + iota16 * 16
                for c in range(3):
                    xr = plsc.load_gather(cl_v, [er * 4 + c])
                    xc = plsc.load_gather(cl_v, [ec * 4 + c])
                    plsc.store_scatter(cdv, [pos + c], xr - xc)

        def write(jj, cdv):
            pltpu.sync_copy(
                cdv, cd_hbm.at[pl.ds((base + jj * CHUNK) * 16, CHUNK * 16)])

        @pl.loop(0, ch, step=2)
        def _(j):
            coord_math(j, cda)
            write(j, cda)
            coord_math(j + 1, cdb)
            write(j + 1, cdb)

    return k(c4flat, rows2d, cols2d)


def _edge_mlp(g1, g2, cd, w1r, We2, be2r, Wc1, bc1r, wc2r, bc2r, n_edges,
              blk):
    e_pad, D = g1.shape
    grid = (e_pad // blk,)

    def body(g1_ref, g2_ref, cd_ref, w1_ref, w2_ref, b2_ref, wc1_ref,
             bc1_ref, wc2_ref, bc2_ref, ef_ref, tr_ref):
        s = g1_ref[...] + g2_ref[...]
        cdv = cd_ref[...]
        radial = jnp.sum(cdv * cdv, axis=1, keepdims=True)
        ef = _silu(s + radial * w1_ref[...])
        edge_feat = _silu(
            jnp.dot(ef.astype(jnp.bfloat16),
                    w2_ref[...].astype(jnp.bfloat16),
                    preferred_element_type=jnp.float32)
            + b2_ref[...])
        tt = _silu(
            jnp.dot(edge_feat.astype(jnp.bfloat16),
                    wc1_ref[...].astype(jnp.bfloat16),
                    preferred_element_type=jnp.float32)
            + bc1_ref[...])
        t = jnp.sum(tt * wc2_ref[...], axis=1, keepdims=True) + bc2_ref[...]
        scale = t / jnp.sqrt(radial + 1e-8)
        trans = cdv * scale
        eid = pl.program_id(0) * blk + lax.broadcasted_iota(
            jnp.int32, (blk, 1), 0)
        valid = eid < n_edges
        ef_ref[...] = jnp.where(valid, edge_feat, 0.0)
        tr_ref[...] = jnp.where(valid, trans, 0.0)

    const = pl.BlockSpec((1, D), lambda i: (0, 0))
    return pl.pallas_call(
        body,
        grid=grid,
        in_specs=[
            pl.BlockSpec((blk, D), lambda i: (i, 0)),
            pl.BlockSpec((blk, D), lambda i: (i, 0)),
            pl.BlockSpec((blk, 16), lambda i: (i, 0)),
            const,
            pl.BlockSpec((D, D), lambda i: (0, 0)),
            const,
            pl.BlockSpec((D, D), lambda i: (0, 0)),
            const,
            const,
            pl.BlockSpec((1, 1), lambda i: (0, 0)),
        ],
        out_specs=[
            pl.BlockSpec((blk, D), lambda i: (i, 0)),
            pl.BlockSpec((blk, 16), lambda i: (i, 0)),
        ],
        out_shape=(
            jax.ShapeDtypeStruct((e_pad, D), jnp.float32),
            jax.ShapeDtypeStruct((e_pad, 16), jnp.float32),
        ),
        compiler_params=pltpu.CompilerParams(
            dimension_semantics=("parallel",)),
    )(g1, g2, cd, w1r, We2, be2r, Wc1, bc1r, wc2r, bc2r)


def _sc_segment_sum_h(ef, rows2d, zh, npad):
    e_pad, D = ef.shape
    ept = e_pad // NW
    ch = ept // CHUNK
    npt = npad // NS
    mesh = plsc.VectorSubcoreMesh(core_axis_name="c", subcore_axis_name="s")

    @functools.partial(
        pl.kernel,
        out_type=jax.ShapeDtypeStruct((NC * npad, D), jnp.float32),
        mesh=mesh,
        scratch_types=[
            pltpu.VMEM_SHARED((npad, D), jnp.float32),
            pltpu.VMEM((ch, CHUNK), jnp.int32),
            pltpu.VMEM((CHUNK, D), jnp.float32),
            pltpu.VMEM((CHUNK, D), jnp.float32),
            pltpu.SemaphoreType.DMA,
            pltpu.SemaphoreType.DMA,
        ],
        compiler_params=_sc_compiler_params(),
    )
    def k(ef_hbm, ri_hbm, zh_hbm, ph_hbm, acc_h, idx_v, vha, vhb, sa, sb):
        cid = lax.axis_index("c")
        sid = lax.axis_index("s")
        wid = cid * NS + sid
        nslc = pl.ds(sid * npt, npt)
        pltpu.sync_copy(zh_hbm, acc_h.at[nslc])
        pltpu.sync_copy(ri_hbm.at[pl.ds(wid * ch, ch)], idx_v)
        plsc.subcore_barrier()
        base = wid * ept

        def src(jj):
            return ef_hbm.at[pl.ds(base + jj * CHUNK, CHUNK)]

        pltpu.async_copy(src(0), vha, sa)

        @pl.loop(0, ch, step=2)
        def _(j):
            pltpu.async_copy(src(j + 1), vhb, sb)
            pltpu.make_async_copy(src(j), vha, sa).wait()
            pltpu.sync_copy(vha, acc_h.at[idx_v.at[j]], add=True)

            @pl.when(j + 2 < ch)
            def _():
                pltpu.async_copy(src(j + 2), vha, sa)

            pltpu.make_async_copy(src(j + 1), vhb, sb).wait()
            pltpu.sync_copy(vhb, acc_h.at[idx_v.at[j + 1]], add=True)

        plsc.subcore_barrier()
        pltpu.sync_copy(acc_h.at[nslc],
                        ph_hbm.at[pl.ds(cid * npad + sid * npt, npt)])

    return k(ef, rows2d, zh)


def _sc_segment_sum_c(trflat, rows2d, zc, npad):
    e_pad16 = trflat.shape[0]
    e_pad = e_pad16 // 16
    ept = e_pad // NW
    ch = ept // CHUNK
    n4 = npad * 4
    mesh = plsc.VectorSubcoreMesh(core_axis_name="c", subcore_axis_name="s")

    @functools.partial(
        pl.kernel,
        out_type=jax.ShapeDtypeStruct((NW * n4,), jnp.float32),
        mesh=mesh,
        scratch_types=[
            pltpu.VMEM((n4,), jnp.float32),
            pltpu.VMEM((ch, CHUNK), jnp.int32),
            pltpu.VMEM((CHUNK * 16,), jnp.float32),
            pltpu.VMEM((CHUNK * 16,), jnp.float32),
            pltpu.SemaphoreType.DMA,
            pltpu.SemaphoreType.DMA,
        ],
        compiler_params=_sc_compiler_params(),
    )
    def k(tr_hbm, ri_hbm, zc_hbm, pc_hbm, acc_c, idx_v, vta, vtb, sa, sb):
        wid = lax.axis_index("c") * NS + lax.axis_index("s")
        pltpu.sync_copy(zc_hbm, acc_c)
        pltpu.sync_copy(ri_hbm.at[pl.ds(wid * ch, ch)], idx_v)
        iota16 = lax.iota(jnp.int32, 16)
        base = wid * ept

        def src(jj):
            return tr_hbm.at[pl.ds((base + jj * CHUNK) * 16, CHUNK * 16)]

        def accumulate(jj, vt_v):
            for sub in range(CHUNK // 16):
                en = idx_v[jj, pl.ds(sub * 16, 16)]
                pos = sub * 256 + iota16 * 16
                for c in range(3):
                    v = plsc.load_gather(vt_v, [pos + c])
                    plsc.addupdate_scatter(acc_c, [en * 4 + c], v)

        pltpu.async_copy(src(0), vta, sa)

        @pl.loop(0, ch, step=2)
        def _(j):
            pltpu.async_copy(src(j + 1), vtb, sb)
            pltpu.make_async_copy(src(j), vta, sa).wait()
            accumulate(j, vta)

            @pl.when(j + 2 < ch)
            def _():
                pltpu.async_copy(src(j + 2), vta, sa)

            pltpu.make_async_copy(src(j + 1), vtb, sb).wait()
            accumulate(j + 1, vtb)

        pltpu.sync_copy(acc_c, pc_hbm.at[pl.ds(wid * n4, n4)])

    return k(trflat, rows2d, zc)


def _node_mlp(h, c4mat, phs, pcmats, Wn1a, Wn1b, bn1r, Wn2, bn2r, npad):
    N, D = h.shape
    rows4 = c4mat.shape[0]
    ns = len(phs)

    def body(h_ref, c_ref, *refs):
        ph_refs = refs[:ns]
        pc_refs = refs[ns:2 * ns]
        wa_ref, wb_ref, b1_ref, w2_ref, b2_ref, ho_ref, co_ref = refs[2 * ns:]
        hh = h_ref[...]
        agg = sum(pr[:N, :] + pr[npad:npad + N, :] for pr in ph_refs)
        m1 = _silu(
            jnp.dot(hh, wa_ref[...], preferred_element_type=jnp.float32)
            + jnp.dot(agg, wb_ref[...], preferred_element_type=jnp.float32)
            + b1_ref[...])
        m = jnp.dot(m1, w2_ref[...], preferred_element_type=jnp.float32)
        ho_ref[...] = hh + m + b2_ref[...]
        aggc = sum(jnp.sum(pr[...], axis=0) for pr in pc_refs)
        co_ref[...] = c_ref[...] + aggc

    return pl.pallas_call(
        body,
        out_shape=(
            jax.ShapeDtypeStruct((N, D), jnp.float32),
            jax.ShapeDtypeStruct((rows4, 128), jnp.float32),
        ),
    )(h, c4mat, *phs, *pcmats, Wn1a, Wn1b, bn1r, Wn2, bn2r)


def kernel(h, edge_index, coord, edge_attr,
           We1, be1, We2, be2, Wn1, bn1, Wn2, bn2, Wc1, bc1, Wc2, bc2):
    del edge_attr
    N, D = h.shape
    E = edge_index.shape[1]
    tile_edges = NW * CHUNK * 8
    e_pad = ((E + tile_edges - 1) // tile_edges) * tile_edges
    npad = ((N + NS * 8 - 1) // (NS * 8)) * (NS * 8)

    row = edge_index[0].astype(jnp.int32)
    col = edge_index[1].astype(jnp.int32)
    rows2d = jnp.pad(row, (0, e_pad - E)).reshape(e_pad // CHUNK, CHUNK)
    cols2d = jnp.pad(col, (0, e_pad - E)).reshape(e_pad // CHUNK, CHUNK)

    c4flat = jnp.pad(coord, ((0, npad - N), (0, 1))).reshape(-1)
    We1a = We1[:D]
    We1b = We1[D:2 * D]
    w1r = We1[2 * D].reshape(1, D)
    be1r = be1.reshape(1, D)
    be2r = be2.reshape(1, D)
    bc1r = bc1.reshape(1, D)
    wc2r = Wc2.reshape(1, D)
    bc2r = bc2.reshape(1, 1)
    bn1r = bn1.reshape(1, D)
    bn2r = bn2.reshape(1, D)
    Wn1a = Wn1[:D]
    Wn1b = Wn1[D:]

    tab = _build_tables(h, We1a, We1b, be1r, npad)
    zh = jnp.zeros((npad // NS, D), jnp.float32)
    zc = jnp.zeros((npad * 4,), jnp.float32)

    S = 2
    es = e_pad // S
    rps = es // CHUNK
    phs, pcs = [], []
    for s in range(S):
        r2d = lax.slice_in_dim(rows2d, s * rps, (s + 1) * rps, axis=0)
        c2d = lax.slice_in_dim(cols2d, s * rps, (s + 1) * rps, axis=0)
        idx2d = jnp.concatenate([r2d, c2d], axis=0)
        g2x = _sc_gather(tab, idx2d, es, npad)
        cd = _sc_coord_diff(c4flat, r2d, c2d, es).reshape(es, 16)
        nval = max(0, min(es, E - s * es))
        ef, tr = _edge_mlp(g2x[:es], g2x[es:], cd, w1r, We2, be2r, Wc1,
                           bc1r, wc2r, bc2r, nval, 2048)
        phs.append(_sc_segment_sum_h(ef, r2d, zh, npad))
        pcs.append(_sc_segment_sum_c(tr.reshape(-1), r2d, zc, npad))

    rows4 = npad * 4 // 128
    pcmats = [pc.reshape(NW, rows4, 128) for pc in pcs]
    c4mat = c4flat.reshape(rows4, 128)
    h_out, co_mat = _node_mlp(h, c4mat, phs, pcmats, Wn1a, Wn1b, bn1r,
                              Wn2, bn2r, npad)
    coord_out = co_mat.reshape(npad, 4)[:N, :3]
    return (h_out, coord_out)


# speedup vs baseline: 4.5907x; 1.0720x over previous
import dataclasses
import functools

import jax
import jax.numpy as jnp
from jax import lax
from jax.experimental import pallas as pl
from jax.experimental.pallas import tpu as pltpu
from jax.experimental.pallas import tpu_sc as plsc

NC = 2
NS = 16
NW = NC * NS
CHUNK = 128


def _sc_compiler_params():
    cp = pltpu.CompilerParams()
    if "needs_layout_passes" in pltpu.CompilerParams.__dataclass_fields__:
        cp = dataclasses.replace(cp, needs_layout_passes=False)
    return cp


def _silu(x):
    return x * jax.nn.sigmoid(x)


def _build_tables(h, We1a, We1b, be1r, npad):
    N, D = h.shape

    def body(h_ref, wa_ref, wb_ref, b1_ref, tab_ref):
        hh = h_ref[...]
        p1 = jnp.dot(hh, wa_ref[...], preferred_element_type=jnp.float32)
        p2 = jnp.dot(hh, wb_ref[...],
                     preferred_element_type=jnp.float32) + b1_ref[...]
        z = jnp.zeros((npad - N, D), jnp.float32)
        tab_ref[...] = jnp.concatenate([p1, z, p2, z], axis=0)

    return pl.pallas_call(
        body,
        out_shape=jax.ShapeDtypeStruct((2 * npad, D), jnp.float32),
    )(h, We1a, We1b, be1r)


def _sc_gather(tab, idx2d, e_pad, npad):
    D = tab.shape[1]
    ept2 = 2 * e_pad // NW
    ch2 = ept2 // CHUNK
    chh = ch2 // 2
    npt = npad // NS
    mesh = plsc.VectorSubcoreMesh(core_axis_name="c", subcore_axis_name="s")

    @functools.partial(
        pl.kernel,
        out_type=jax.ShapeDtypeStruct((2 * e_pad, D), jnp.float32),
        mesh=mesh,
        scratch_types=[
            pltpu.VMEM_SHARED((npad, D), jnp.float32),
            pltpu.VMEM((chh, CHUNK), jnp.int32),
            pltpu.VMEM((CHUNK, D), jnp.float32),
            pltpu.VMEM((CHUNK, D), jnp.float32),
            pltpu.SemaphoreType.DMA,
            pltpu.SemaphoreType.DMA,
        ],
        compiler_params=_sc_compiler_params(),
    )
    def k(tab_hbm, ix_hbm, g_hbm, spm, ix_v, ba, bb, sa, sb):
        cid = lax.axis_index("c")
        sid = lax.axis_index("s")
        wid = cid * NS + sid
        pltpu.sync_copy(tab_hbm.at[pl.ds(cid * npad + sid * npt, npt)],
                        spm.at[pl.ds(sid * npt, npt)])
        plsc.subcore_barrier()
        base = wid * ept2

        def issue(jj, buf, s):
            pltpu.async_copy(spm.at[ix_v.at[jj]], buf, s)

        def finish(jj, half, buf, s):
            pltpu.make_async_copy(spm.at[ix_v.at[jj]], buf, s).wait()
            pltpu.sync_copy(
                buf,
                g_hbm.at[pl.ds(base + (half * chh + jj) * CHUNK, CHUNK)])

        for half in range(2):
            pltpu.sync_copy(
                ix_hbm.at[pl.ds(wid * ch2 + half * chh, chh)], ix_v)
            issue(0, ba, sa)

            @pl.loop(0, chh, step=2)
            def _(j):
                issue(j + 1, bb, sb)
                finish(j, half, ba, sa)

                @pl.when(j + 2 < chh)
                def _():
                    issue(j + 2, ba, sa)

                finish(j + 1, half, bb, sb)

    return k(tab, idx2d)


def _sc_coord_diff(c4flat, rows2d, cols2d, e_pad):
    n4 = c4flat.shape[0]
    ept = e_pad // NW
    ch = ept // CHUNK
    mesh = plsc.VectorSubcoreMesh(core_axis_name="c", subcore_axis_name="s")

    @functools.partial(
        pl.kernel,
        out_type=jax.ShapeDtypeStruct((e_pad * 16,), jnp.float32),
        mesh=mesh,
        scratch_types=[
            pltpu.VMEM((ch, CHUNK), jnp.int32),
            pltpu.VMEM((ch, CHUNK), jnp.int32),
            pltpu.VMEM((n4,), jnp.float32),
            pltpu.VMEM((CHUNK * 16,), jnp.float32),
            pltpu.VMEM((CHUNK * 16,), jnp.float32),
        ],
        compiler_params=_sc_compiler_params(),
    )
    def k(c4_hbm, ri_hbm, ci_hbm, cd_hbm, ir_v, ic_v, cl_v, cda, cdb):
        wid = lax.axis_index("c") * NS + lax.axis_index("s")
        pltpu.sync_copy(ri_hbm.at[pl.ds(wid * ch, ch)], ir_v)
        pltpu.sync_copy(ci_hbm.at[pl.ds(wid * ch, ch)], ic_v)
        pltpu.sync_copy(c4_hbm, cl_v)

        zero16 = jnp.zeros((16,), jnp.float32)

        @pl.loop(0, CHUNK * 16, step=16)
        def _(i):
            cda[pl.ds(i, 16)] = zero16
            cdb[pl.ds(i, 16)] = zero16

        iota16 = lax.iota(jnp.int32, 16)
        base = wid * ept

        def coord_math(jj, cdv):
            for sub in range(CHUNK // 16):
                er = ir_v[jj, pl.ds(sub * 16, 16)]
                ec = ic_v[jj, pl.ds(sub * 16, 16)]
                pos = sub * 256 + iota16 * 16
                for c in range(3):
                    xr = plsc.load_gather(cl_v, [er * 4 + c])
                    xc = plsc.load_gather(cl_v, [ec * 4 + c])
                    plsc.store_scatter(cdv, [pos + c], xr - xc)

        def write(jj, cdv):
            pltpu.sync_copy(
                cdv, cd_hbm.at[pl.ds((base + jj * CHUNK) * 16, CHUNK * 16)])

        @pl.loop(0, ch, step=2)
        def _(j):
            coord_math(j, cda)
            write(j, cda)
            coord_math(j + 1, cdb)
            write(j + 1, cdb)

    return k(c4flat, rows2d, cols2d)


def _edge_mlp(g1, g2, cd, w1r, We2, be2r, Wc1, bc1r, wc2r, bc2r, n_edges,
              blk):
    e_pad, D = g1.shape
    grid = (e_pad // blk,)

    def body(g1_ref, g2_ref, cd_ref, w1_ref, w2_ref, b2_ref, wc1_ref,
             bc1_ref, wc2_ref, bc2_ref, ef_ref, tr_ref):
        s = g1_ref[...] + g2_ref[...]
        cdv = cd_ref[...]
        radial = jnp.sum(cdv * cdv, axis=1, keepdims=True)
        ef = _silu((s + radial * w1_ref[...]).astype(jnp.bfloat16))
        edge_feat = _silu(
            (jnp.dot(ef, w2_ref[...].astype(jnp.bfloat16),
                     preferred_element_type=jnp.float32)
             + b2_ref[...]).astype(jnp.bfloat16))
        tt = _silu(
            (jnp.dot(edge_feat, wc1_ref[...].astype(jnp.bfloat16),
                     preferred_element_type=jnp.float32)
             + bc1_ref[...]).astype(jnp.bfloat16))
        t = jnp.dot(tt, wc2_ref[...].astype(jnp.bfloat16),
                    preferred_element_type=jnp.float32)[:, 0:1] + bc2_ref[...]
        scale = t / jnp.sqrt(radial + 1e-8)
        trans = cdv * scale
        eid = pl.program_id(0) * blk + lax.broadcasted_iota(
            jnp.int32, (blk, 1), 0)
        valid = eid < n_edges
        ef_ref[...] = jnp.where(valid, edge_feat.astype(jnp.float32), 0.0)
        tr_ref[...] = jnp.where(valid, trans, 0.0)

    const = pl.BlockSpec((1, D), lambda i: (0, 0))
    return pl.pallas_call(
        body,
        grid=grid,
        in_specs=[
            pl.BlockSpec((blk, D), lambda i: (i, 0)),
            pl.BlockSpec((blk, D), lambda i: (i, 0)),
            pl.BlockSpec((blk, 16), lambda i: (i, 0)),
            const,
            pl.BlockSpec((D, D), lambda i: (0, 0)),
            const,
            pl.BlockSpec((D, D), lambda i: (0, 0)),
            const,
            pl.BlockSpec((D, D), lambda i: (0, 0)),
            pl.BlockSpec((1, 1), lambda i: (0, 0)),
        ],
        out_specs=[
            pl.BlockSpec((blk, D), lambda i: (i, 0)),
            pl.BlockSpec((blk, 16), lambda i: (i, 0)),
        ],
        out_shape=(
            jax.ShapeDtypeStruct((e_pad, D), jnp.float32),
            jax.ShapeDtypeStruct((e_pad, 16), jnp.float32),
        ),
        compiler_params=pltpu.CompilerParams(
            dimension_semantics=("parallel",)),
    )(g1, g2, cd, w1r, We2, be2r, Wc1, bc1r, wc2r, bc2r)


def _sc_segment_sum_h(ef, rows2d, zh, npad):
    e_pad, D = ef.shape
    ept = e_pad // NW
    ch = ept // CHUNK
    npt = npad // NS
    mesh = plsc.VectorSubcoreMesh(core_axis_name="c", subcore_axis_name="s")

    @functools.partial(
        pl.kernel,
        out_type=jax.ShapeDtypeStruct((NC * npad, D), jnp.float32),
        mesh=mesh,
        scratch_types=[
            pltpu.VMEM_SHARED((npad, D), jnp.float32),
            pltpu.VMEM((ch, CHUNK), jnp.int32),
            pltpu.VMEM((CHUNK, D), jnp.float32),
            pltpu.VMEM((CHUNK, D), jnp.float32),
            pltpu.SemaphoreType.DMA,
            pltpu.SemaphoreType.DMA,
        ],
        compiler_params=_sc_compiler_params(),
    )
    def k(ef_hbm, ri_hbm, zh_hbm, ph_hbm, acc_h, idx_v, vha, vhb, sa, sb):
        cid = lax.axis_index("c")
        sid = lax.axis_index("s")
        wid = cid * NS + sid
        nslc = pl.ds(sid * npt, npt)
        pltpu.sync_copy(zh_hbm, acc_h.at[nslc])
        pltpu.sync_copy(ri_hbm.at[pl.ds(wid * ch, ch)], idx_v)
        plsc.subcore_barrier()
        base = wid * ept

        def src(jj):
            return ef_hbm.at[pl.ds(base + jj * CHUNK, CHUNK)]

        pltpu.async_copy(src(0), vha, sa)

        @pl.loop(0, ch, step=2)
        def _(j):
            pltpu.async_copy(src(j + 1), vhb, sb)
            pltpu.make_async_copy(src(j), vha, sa).wait()
            pltpu.sync_copy(vha, acc_h.at[idx_v.at[j]], add=True)

            @pl.when(j + 2 < ch)
            def _():
                pltpu.async_copy(src(j + 2), vha, sa)

            pltpu.make_async_copy(src(j + 1), vhb, sb).wait()
            pltpu.sync_copy(vhb, acc_h.at[idx_v.at[j + 1]], add=True)

        plsc.subcore_barrier()
        pltpu.sync_copy(acc_h.at[nslc],
                        ph_hbm.at[pl.ds(cid * npad + sid * npt, npt)])

    return k(ef, rows2d, zh)


def _sc_segment_sum_c(trflat, rows2d, zc, npad):
    e_pad16 = trflat.shape[0]
    e_pad = e_pad16 // 16
    ept = e_pad // NW
    ch = ept // CHUNK
    n4 = npad * 4
    mesh = plsc.VectorSubcoreMesh(core_axis_name="c", subcore_axis_name="s")

    @functools.partial(
        pl.kernel,
        out_type=jax.ShapeDtypeStruct((NW * n4,), jnp.float32),
        mesh=mesh,
        scratch_types=[
            pltpu.VMEM((n4,), jnp.float32),
            pltpu.VMEM((ch, CHUNK), jnp.int32),
            pltpu.VMEM((CHUNK * 16,), jnp.float32),
            pltpu.VMEM((CHUNK * 16,), jnp.float32),
            pltpu.SemaphoreType.DMA,
            pltpu.SemaphoreType.DMA,
        ],
        compiler_params=_sc_compiler_params(),
    )
    def k(tr_hbm, ri_hbm, zc_hbm, pc_hbm, acc_c, idx_v, vta, vtb, sa, sb):
        wid = lax.axis_index("c") * NS + lax.axis_index("s")
        pltpu.sync_copy(zc_hbm, acc_c)
        pltpu.sync_copy(ri_hbm.at[pl.ds(wid * ch, ch)], idx_v)
        iota16 = lax.iota(jnp.int32, 16)
        base = wid * ept

        def src(jj):
            return tr_hbm.at[pl.ds((base + jj * CHUNK) * 16, CHUNK * 16)]

        def accumulate(jj, vt_v):
            for sub in range(CHUNK // 16):
                en = idx_v[jj, pl.ds(sub * 16, 16)]
                pos = sub * 256 + iota16 * 16
                for c in range(3):
                    v = plsc.load_gather(vt_v, [pos + c])
                    plsc.addupdate_scatter(acc_c, [en * 4 + c], v)

        pltpu.async_copy(src(0), vta, sa)

        @pl.loop(0, ch, step=2)
        def _(j):
            pltpu.async_copy(src(j + 1), vtb, sb)
            pltpu.make_async_copy(src(j), vta, sa).wait()
            accumulate(j, vta)

            @pl.when(j + 2 < ch)
            def _():
                pltpu.async_copy(src(j + 2), vta, sa)

            pltpu.make_async_copy(src(j + 1), vtb, sb).wait()
            accumulate(j + 1, vtb)

        pltpu.sync_copy(acc_c, pc_hbm.at[pl.ds(wid * n4, n4)])

    return k(trflat, rows2d, zc)


def _node_mlp(h, c4mat, phs, pcmats, Wn1a, Wn1b, bn1r, Wn2, bn2r, npad):
    N, D = h.shape
    rows4 = c4mat.shape[0]
    ns = len(phs)

    def body(h_ref, c_ref, *refs):
        ph_refs = refs[:ns]
        pc_refs = refs[ns:2 * ns]
        wa_ref, wb_ref, b1_ref, w2_ref, b2_ref, ho_ref, co_ref = refs[2 * ns:]
        hh = h_ref[...]
        agg = sum(pr[:N, :] + pr[npad:npad + N, :] for pr in ph_refs)
        m1 = _silu(
            jnp.dot(hh, wa_ref[...], preferred_element_type=jnp.float32)
            + jnp.dot(agg, wb_ref[...], preferred_element_type=jnp.float32)
            + b1_ref[...])
        m = jnp.dot(m1, w2_ref[...], preferred_element_type=jnp.float32)
        ho_ref[...] = hh + m + b2_ref[...]
        aggc = sum(jnp.sum(pr[...], axis=0) for pr in pc_refs)
        co_ref[...] = c_ref[...] + aggc

    return pl.pallas_call(
        body,
        out_shape=(
            jax.ShapeDtypeStruct((N, D), jnp.float32),
            jax.ShapeDtypeStruct((rows4, 128), jnp.float32),
        ),
    )(h, c4mat, *phs, *pcmats, Wn1a, Wn1b, bn1r, Wn2, bn2r)


def kernel(h, edge_index, coord, edge_attr,
           We1, be1, We2, be2, Wn1, bn1, Wn2, bn2, Wc1, bc1, Wc2, bc2):
    del edge_attr
    N, D = h.shape
    E = edge_index.shape[1]
    tile_edges = NW * CHUNK * 8
    e_pad = ((E + tile_edges - 1) // tile_edges) * tile_edges
    npad = ((N + NS * 8 - 1) // (NS * 8)) * (NS * 8)

    row = edge_index[0].astype(jnp.int32)
    col = edge_index[1].astype(jnp.int32)
    rows2d = jnp.pad(row, (0, e_pad - E)).reshape(e_pad // CHUNK, CHUNK)
    cols2d = jnp.pad(col, (0, e_pad - E)).reshape(e_pad // CHUNK, CHUNK)

    c4flat = jnp.pad(coord, ((0, npad - N), (0, 1))).reshape(-1)
    We1a = We1[:D]
    We1b = We1[D:2 * D]
    w1r = We1[2 * D].reshape(1, D)
    be1r = be1.reshape(1, D)
    be2r = be2.reshape(1, D)
    bc1r = bc1.reshape(1, D)
    wc2r = jnp.pad(Wc2, ((0, 0), (0, D - 1)))
    bc2r = bc2.reshape(1, 1)
    bn1r = bn1.reshape(1, D)
    bn2r = bn2.reshape(1, D)
    Wn1a = Wn1[:D]
    Wn1b = Wn1[D:]

    tab = _build_tables(h, We1a, We1b, be1r, npad)
    zh = jnp.zeros((npad // NS, D), jnp.float32)
    zc = jnp.zeros((npad * 4,), jnp.float32)

    S = 2
    es = e_pad // S
    rps = es // CHUNK
    phs, pcs = [], []
    for s in range(S):
        r2d = lax.slice_in_dim(rows2d, s * rps, (s + 1) * rps, axis=0)
        c2d = lax.slice_in_dim(cols2d, s * rps, (s + 1) * rps, axis=0)
        idx2d = jnp.concatenate([r2d, c2d], axis=0)
        g2x = _sc_gather(tab, idx2d, es, npad)
        cd = _sc_coord_diff(c4flat, r2d, c2d, es).reshape(es, 16)
        nval = max(0, min(es, E - s * es))
        ef, tr = _edge_mlp(g2x[:es], g2x[es:], cd, w1r, We2, be2r, Wc1,
                           bc1r, wc2r, bc2r, nval, 4096)
        phs.append(_sc_segment_sum_h(ef, r2d, zh, npad))
        pcs.append(_sc_segment_sum_c(tr.reshape(-1), r2d, zc, npad))

    rows4 = npad * 4 // 128
    pcmats = [pc.reshape(NW, rows4, 128) for pc in pcs]
    c4mat = c4flat.reshape(rows4, 128)
    h_out, co_mat = _node_mlp(h, c4mat, phs, pcmats, Wn1a, Wn1b, bn1r,
                              Wn2, bn2r, npad)
    coord_out = co_mat.reshape(npad, 4)[:N, :3]
    return (h_out, coord_out)


# speedup vs baseline: 4.6052x; 1.0032x over previous
import dataclasses
import functools

import jax
import jax.numpy as jnp
from jax import lax
from jax.experimental import pallas as pl
from jax.experimental.pallas import tpu as pltpu
from jax.experimental.pallas import tpu_sc as plsc

NC = 2
NS = 16
NW = NC * NS
CHUNK = 128


def _sc_compiler_params():
    cp = pltpu.CompilerParams()
    if "needs_layout_passes" in pltpu.CompilerParams.__dataclass_fields__:
        cp = dataclasses.replace(cp, needs_layout_passes=False)
    return cp


def _silu(x):
    return x * jax.nn.sigmoid(x)


def _build_tables(h, We1a, We1b, be1r, npad):
    N, D = h.shape

    def body(h_ref, wa_ref, wb_ref, b1_ref, tab_ref):
        hh = h_ref[...]
        p1 = jnp.dot(hh, wa_ref[...], preferred_element_type=jnp.float32)
        p2 = jnp.dot(hh, wb_ref[...],
                     preferred_element_type=jnp.float32) + b1_ref[...]
        z = jnp.zeros((npad - N, D), jnp.float32)
        tab_ref[...] = jnp.concatenate([p1, z, p2, z], axis=0)

    return pl.pallas_call(
        body,
        out_shape=jax.ShapeDtypeStruct((2 * npad, D), jnp.float32),
    )(h, We1a, We1b, be1r)


def _sc_gather(tab, idx2d, e_pad, npad):
    D = tab.shape[1]
    ept2 = 2 * e_pad // NW
    ch2 = ept2 // CHUNK
    chh = ch2 // 2
    npt = npad // NS
    mesh = plsc.VectorSubcoreMesh(core_axis_name="c", subcore_axis_name="s")

    @functools.partial(
        pl.kernel,
        out_type=jax.ShapeDtypeStruct((2 * e_pad, D), jnp.float32),
        mesh=mesh,
        scratch_types=[
            pltpu.VMEM_SHARED((npad, D), jnp.float32),
            pltpu.VMEM((chh, CHUNK), jnp.int32),
            pltpu.VMEM((CHUNK, D), jnp.float32),
            pltpu.VMEM((CHUNK, D), jnp.float32),
            pltpu.SemaphoreType.DMA,
            pltpu.SemaphoreType.DMA,
        ],
        compiler_params=_sc_compiler_params(),
    )
    def k(tab_hbm, ix_hbm, g_hbm, spm, ix_v, ba, bb, sa, sb):
        cid = lax.axis_index("c")
        sid = lax.axis_index("s")
        wid = cid * NS + sid
        pltpu.sync_copy(tab_hbm.at[pl.ds(cid * npad + sid * npt, npt)],
                        spm.at[pl.ds(sid * npt, npt)])
        plsc.subcore_barrier()
        base = wid * ept2

        def issue(jj, buf, s):
            pltpu.async_copy(spm.at[ix_v.at[jj]], buf, s)

        def finish(jj, half, buf, s):
            pltpu.make_async_copy(spm.at[ix_v.at[jj]], buf, s).wait()
            pltpu.sync_copy(
                buf,
                g_hbm.at[pl.ds(base + (half * chh + jj) * CHUNK, CHUNK)])

        for half in range(2):
            pltpu.sync_copy(
                ix_hbm.at[pl.ds(wid * ch2 + half * chh, chh)], ix_v)
            issue(0, ba, sa)

            @pl.loop(0, chh, step=2)
            def _(j):
                issue(j + 1, bb, sb)
                finish(j, half, ba, sa)

                @pl.when(j + 2 < chh)
                def _():
                    issue(j + 2, ba, sa)

                finish(j + 1, half, bb, sb)

    return k(tab, idx2d)


def _sc_coord_diff(c4flat, rows2d, cols2d, e_pad):
    n4 = c4flat.shape[0]
    ept = e_pad // NW
    ch = ept // CHUNK
    mesh = plsc.VectorSubcoreMesh(core_axis_name="c", subcore_axis_name="s")

    @functools.partial(
        pl.kernel,
        out_type=jax.ShapeDtypeStruct((e_pad * 16,), jnp.float32),
        mesh=mesh,
        scratch_types=[
            pltpu.VMEM((ch, CHUNK), jnp.int32),
            pltpu.VMEM((ch, CHUNK), jnp.int32),
            pltpu.VMEM((n4,), jnp.float32),
            pltpu.VMEM((CHUNK * 16,), jnp.float32),
            pltpu.VMEM((CHUNK * 16,), jnp.float32),
        ],
        compiler_params=_sc_compiler_params(),
    )
    def k(c4_hbm, ri_hbm, ci_hbm, cd_hbm, ir_v, ic_v, cl_v, cda, cdb):
        wid = lax.axis_index("c") * NS + lax.axis_index("s")
        pltpu.sync_copy(ri_hbm.at[pl.ds(wid * ch, ch)], ir_v)
        pltpu.sync_copy(ci_hbm.at[pl.ds(wid * ch, ch)], ic_v)
        pltpu.sync_copy(c4_hbm, cl_v)

        zero16 = jnp.zeros((16,), jnp.float32)

        @pl.loop(0, CHUNK * 16, step=16)
        def _(i):
            cda[pl.ds(i, 16)] = zero16
            cdb[pl.ds(i, 16)] = zero16

        iota16 = lax.iota(jnp.int32, 16)
        base = wid * ept

        def coord_math(jj, cdv):
            for sub in range(CHUNK // 16):
                er = ir_v[jj, pl.ds(sub * 16, 16)]
                ec = ic_v[jj, pl.ds(sub * 16, 16)]
                pos = sub * 256 + iota16 * 16
                for c in range(3):
                    xr = plsc.load_gather(cl_v, [er * 4 + c])
                    xc = plsc.load_gather(cl_v, [ec * 4 + c])
                    plsc.store_scatter(cdv, [pos + c], xr - xc)

        def write(jj, cdv):
            pltpu.sync_copy(
                cdv, cd_hbm.at[pl.ds((base + jj * CHUNK) * 16, CHUNK * 16)])

        @pl.loop(0, ch, step=2)
        def _(j):
            coord_math(j, cda)
            write(j, cda)
            coord_math(j + 1, cdb)
            write(j + 1, cdb)

    return k(c4flat, rows2d, cols2d)


def _edge_mlp(g1, g2, cd, w1r, We2, be2r, Wc1, bc1r, wc2r, bc2r, n_edges,
              blk):
    e_pad, D = g1.shape
    grid = (e_pad // blk,)

    def body(g1_ref, g2_ref, cd_ref, w1_ref, w2_ref, b2_ref, wc1_ref,
             bc1_ref, wc2_ref, bc2_ref, ef_ref, tr_ref):
        s = g1_ref[...] + g2_ref[...]
        cdv = cd_ref[...]
        radial = jnp.sum(cdv * cdv, axis=1, keepdims=True)
        ef = _silu((s + radial * w1_ref[...]).astype(jnp.bfloat16))
        edge_feat = _silu(
            (jnp.dot(ef, w2_ref[...].astype(jnp.bfloat16),
                     preferred_element_type=jnp.float32)
             + b2_ref[...]).astype(jnp.bfloat16))
        tt = _silu(
            (jnp.dot(edge_feat, wc1_ref[...].astype(jnp.bfloat16),
                     preferred_element_type=jnp.float32)
             + bc1_ref[...]).astype(jnp.bfloat16))
        t = jnp.dot(tt, wc2_ref[...].astype(jnp.bfloat16),
                    preferred_element_type=jnp.float32)[:, 0:1] + bc2_ref[...]
        scale = t / jnp.sqrt(radial + 1e-8)
        trans = cdv * scale
        eid = pl.program_id(0) * blk + lax.broadcasted_iota(
            jnp.int32, (blk, 1), 0)
        valid = eid < n_edges
        ef_ref[...] = jnp.where(valid, edge_feat.astype(jnp.float32), 0.0)
        tr_ref[...] = jnp.where(valid, trans, 0.0)

    const = pl.BlockSpec((1, D), lambda i: (0, 0))
    return pl.pallas_call(
        body,
        grid=grid,
        in_specs=[
            pl.BlockSpec((blk, D), lambda i: (i, 0)),
            pl.BlockSpec((blk, D), lambda i: (i, 0)),
            pl.BlockSpec((blk, 16), lambda i: (i, 0)),
            const,
            pl.BlockSpec((D, D), lambda i: (0, 0)),
            const,
            pl.BlockSpec((D, D), lambda i: (0, 0)),
            const,
            pl.BlockSpec((D, D), lambda i: (0, 0)),
            pl.BlockSpec((1, 1), lambda i: (0, 0)),
        ],
        out_specs=[
            pl.BlockSpec((blk, D), lambda i: (i, 0)),
            pl.BlockSpec((blk, 16), lambda i: (i, 0)),
        ],
        out_shape=(
            jax.ShapeDtypeStruct((e_pad, D), jnp.float32),
            jax.ShapeDtypeStruct((e_pad, 16), jnp.float32),
        ),
        compiler_params=pltpu.CompilerParams(
            dimension_semantics=("parallel",)),
    )(g1, g2, cd, w1r, We2, be2r, Wc1, bc1r, wc2r, bc2r)


def _sc_segment_sum_h(ef, rows2d, zh, npad):
    e_pad, D = ef.shape
    ept = e_pad // NW
    ch = ept // CHUNK
    npt = npad // NS
    mesh = plsc.VectorSubcoreMesh(core_axis_name="c", subcore_axis_name="s")

    @functools.partial(
        pl.kernel,
        out_type=jax.ShapeDtypeStruct((NC * npad, D), jnp.float32),
        mesh=mesh,
        scratch_types=[
            pltpu.VMEM_SHARED((npad, D), jnp.float32),
            pltpu.VMEM((ch, CHUNK), jnp.int32),
            pltpu.VMEM((CHUNK, D), jnp.float32),
            pltpu.VMEM((CHUNK, D), jnp.float32),
            pltpu.SemaphoreType.DMA,
            pltpu.SemaphoreType.DMA,
        ],
        compiler_params=_sc_compiler_params(),
    )
    def k(ef_hbm, ri_hbm, zh_hbm, ph_hbm, acc_h, idx_v, vha, vhb, sa, sb):
        cid = lax.axis_index("c")
        sid = lax.axis_index("s")
        wid = cid * NS + sid
        nslc = pl.ds(sid * npt, npt)
        pltpu.sync_copy(zh_hbm, acc_h.at[nslc])
        pltpu.sync_copy(ri_hbm.at[pl.ds(wid * ch, ch)], idx_v)
        plsc.subcore_barrier()
        base = wid * ept

        def src(jj):
            return ef_hbm.at[pl.ds(base + jj * CHUNK, CHUNK)]

        pltpu.async_copy(src(0), vha, sa)

        @pl.loop(0, ch, step=2)
        def _(j):
            pltpu.async_copy(src(j + 1), vhb, sb)
            pltpu.make_async_copy(src(j), vha, sa).wait()
            pltpu.sync_copy(vha, acc_h.at[idx_v.at[j]], add=True)

            @pl.when(j + 2 < ch)
            def _():
                pltpu.async_copy(src(j + 2), vha, sa)

            pltpu.make_async_copy(src(j + 1), vhb, sb).wait()
            pltpu.sync_copy(vhb, acc_h.at[idx_v.at[j + 1]], add=True)

        plsc.subcore_barrier()
        pltpu.sync_copy(acc_h.at[nslc],
                        ph_hbm.at[pl.ds(cid * npad + sid * npt, npt)])

    return k(ef, rows2d, zh)


def _sc_segment_sum_c(trflat, rows2d, zc, npad):
    e_pad16 = trflat.shape[0]
    e_pad = e_pad16 // 16
    ept = e_pad // NW
    ch = ept // CHUNK
    n4 = npad * 4
    mesh = plsc.VectorSubcoreMesh(core_axis_name="c", subcore_axis_name="s")

    @functools.partial(
        pl.kernel,
        out_type=jax.ShapeDtypeStruct((NW * n4,), jnp.float32),
        mesh=mesh,
        scratch_types=[
            pltpu.VMEM((n4,), jnp.float32),
            pltpu.VMEM((ch, CHUNK), jnp.int32),
            pltpu.VMEM((CHUNK * 16,), jnp.float32),
            pltpu.VMEM((CHUNK * 16,), jnp.float32),
            pltpu.SemaphoreType.DMA,
            pltpu.SemaphoreType.DMA,
        ],
        compiler_params=_sc_compiler_params(),
    )
    def k(tr_hbm, ri_hbm, zc_hbm, pc_hbm, acc_c, idx_v, vta, vtb, sa, sb):
        wid = lax.axis_index("c") * NS + lax.axis_index("s")
        pltpu.sync_copy(zc_hbm, acc_c)
        pltpu.sync_copy(ri_hbm.at[pl.ds(wid * ch, ch)], idx_v)
        iota16 = lax.iota(jnp.int32, 16)
        base = wid * ept

        def src(jj):
            return tr_hbm.at[pl.ds((base + jj * CHUNK) * 16, CHUNK * 16)]

        def accumulate(jj, vt_v):
            for sub in range(CHUNK // 16):
                en = idx_v[jj, pl.ds(sub * 16, 16)]
                pos = sub * 256 + iota16 * 16
                for c in range(3):
                    v = plsc.load_gather(vt_v, [pos + c])
                    plsc.addupdate_scatter(acc_c, [en * 4 + c], v)

        pltpu.async_copy(src(0), vta, sa)

        @pl.loop(0, ch, step=2)
        def _(j):
            pltpu.async_copy(src(j + 1), vtb, sb)
            pltpu.make_async_copy(src(j), vta, sa).wait()
            accumulate(j, vta)

            @pl.when(j + 2 < ch)
            def _():
                pltpu.async_copy(src(j + 2), vta, sa)

            pltpu.make_async_copy(src(j + 1), vtb, sb).wait()
            accumulate(j + 1, vtb)

        pltpu.sync_copy(acc_c, pc_hbm.at[pl.ds(wid * n4, n4)])

    return k(trflat, rows2d, zc)


def _node_mlp(h, c4mat, phs, pcmats, Wn1a, Wn1b, bn1r, Wn2, bn2r, npad):
    N, D = h.shape
    rows4 = c4mat.shape[0]
    ns = len(phs)

    def body(h_ref, c_ref, *refs):
        ph_refs = refs[:ns]
        pc_refs = refs[ns:2 * ns]
        wa_ref, wb_ref, b1_ref, w2_ref, b2_ref, ho_ref, co_ref = refs[2 * ns:]
        hh = h_ref[...]
        agg = sum(pr[:N, :] + pr[npad:npad + N, :] for pr in ph_refs)
        m1 = _silu(
            jnp.dot(hh, wa_ref[...], preferred_element_type=jnp.float32)
            + jnp.dot(agg, wb_ref[...], preferred_element_type=jnp.float32)
            + b1_ref[...])
        m = jnp.dot(m1, w2_ref[...], preferred_element_type=jnp.float32)
        ho_ref[...] = hh + m + b2_ref[...]
        aggc = sum(jnp.sum(pr[...], axis=0) for pr in pc_refs)
        co_ref[...] = c_ref[...] + aggc

    return pl.pallas_call(
        body,
        out_shape=(
            jax.ShapeDtypeStruct((N, D), jnp.float32),
            jax.ShapeDtypeStruct((rows4, 128), jnp.float32),
        ),
    )(h, c4mat, *phs, *pcmats, Wn1a, Wn1b, bn1r, Wn2, bn2r)


def kernel(h, edge_index, coord, edge_attr,
           We1, be1, We2, be2, Wn1, bn1, Wn2, bn2, Wc1, bc1, Wc2, bc2):
    del edge_attr
    N, D = h.shape
    E = edge_index.shape[1]
    tile_edges = NW * CHUNK * 8
    e_pad = ((E + tile_edges - 1) // tile_edges) * tile_edges
    npad = ((N + NS * 8 - 1) // (NS * 8)) * (NS * 8)

    row = edge_index[0].astype(jnp.int32)
    col = edge_index[1].astype(jnp.int32)
    rows2d = jnp.pad(row, (0, e_pad - E)).reshape(e_pad // CHUNK, CHUNK)
    cols2d = jnp.pad(col, (0, e_pad - E)).reshape(e_pad // CHUNK, CHUNK)

    c4flat = jnp.pad(coord, ((0, npad - N), (0, 1))).reshape(-1)
    We1a = We1[:D]
    We1b = We1[D:2 * D]
    w1r = We1[2 * D].reshape(1, D)
    be1r = be1.reshape(1, D)
    be2r = be2.reshape(1, D)
    bc1r = bc1.reshape(1, D)
    wc2r = jnp.pad(Wc2, ((0, 0), (0, D - 1)))
    bc2r = bc2.reshape(1, 1)
    bn1r = bn1.reshape(1, D)
    bn2r = bn2.reshape(1, D)
    Wn1a = Wn1[:D]
    Wn1b = Wn1[D:]

    tab = _build_tables(h, We1a, We1b, be1r, npad)
    zh = jnp.zeros((npad // NS, D), jnp.float32)
    zc = jnp.zeros((npad * 4,), jnp.float32)

    S = 1
    es = e_pad // S
    rps = es // CHUNK
    phs, pcs = [], []
    for s in range(S):
        r2d = lax.slice_in_dim(rows2d, s * rps, (s + 1) * rps, axis=0)
        c2d = lax.slice_in_dim(cols2d, s * rps, (s + 1) * rps, axis=0)
        idx2d = jnp.concatenate([r2d, c2d], axis=0)
        g2x = _sc_gather(tab, idx2d, es, npad)
        cd = _sc_coord_diff(c4flat, r2d, c2d, es).reshape(es, 16)
        nval = max(0, min(es, E - s * es))
        ef, tr = _edge_mlp(g2x[:es], g2x[es:], cd, w1r, We2, be2r, Wc1,
                           bc1r, wc2r, bc2r, nval, 4096)
        phs.append(_sc_segment_sum_h(ef, r2d, zh, npad))
        pcs.append(_sc_segment_sum_c(tr.reshape(-1), r2d, zc, npad))

    rows4 = npad * 4 // 128
    pcmats = [pc.reshape(NW, rows4, 128) for pc in pcs]
    c4mat = c4flat.reshape(rows4, 128)
    h_out, co_mat = _node_mlp(h, c4mat, phs, pcmats, Wn1a, Wn1b, bn1r,
                              Wn2, bn2r, npad)
    coord_out = co_mat.reshape(npad, 4)[:N, :3]
    return (h_out, coord_out)


# speedup vs baseline: 6.6056x; 1.4344x over previous
import dataclasses
import functools

import jax
import jax.numpy as jnp
from jax import lax
from jax.experimental import pallas as pl
from jax.experimental.pallas import tpu as pltpu
from jax.experimental.pallas import tpu_sc as plsc

NC = 2
NS = 16
NW = NC * NS
CHUNK = 128


def _sc_compiler_params():
    cp = pltpu.CompilerParams()
    if "needs_layout_passes" in pltpu.CompilerParams.__dataclass_fields__:
        cp = dataclasses.replace(cp, needs_layout_passes=False)
    return cp


def _silu(x):
    return x * jax.nn.sigmoid(x)


def _pack_bf16_pair(x):
    hlf = x.shape[1] // 2
    lo = lax.bitcast_convert_type(
        x[:, :hlf].astype(jnp.bfloat16), jnp.uint16).astype(jnp.uint32)
    hi = lax.bitcast_convert_type(
        x[:, hlf:].astype(jnp.bfloat16), jnp.uint16).astype(jnp.uint32)
    return lax.bitcast_convert_type(lo | (hi << 16), jnp.float32)


def _build_tables(h, c16, We1a, We1b, be1r, npad):
    N, D = h.shape

    def body(h_ref, c_ref, wa_ref, wb_ref, b1_ref, tab_ref):
        hh = h_ref[...]
        cc = c_ref[...]
        p1 = jnp.dot(hh, wa_ref[...], preferred_element_type=jnp.float32)
        p2 = jnp.dot(hh, wb_ref[...],
                     preferred_element_type=jnp.float32) + b1_ref[...]
        z48 = jnp.zeros((N, D - D // 2 - 16), jnp.float32)
        r1 = jnp.concatenate([_pack_bf16_pair(p1), cc, z48], axis=1)
        r2 = jnp.concatenate([_pack_bf16_pair(p2), cc, z48], axis=1)
        z = jnp.zeros((npad - N, D), jnp.float32)
        tab_ref[...] = jnp.concatenate([r1, z, r2, z], axis=0)

    return pl.pallas_call(
        body,
        out_shape=jax.ShapeDtypeStruct((2 * npad, D), jnp.float32),
    )(h, c16, We1a, We1b, be1r)


def _sc_gather(tab, idx2d, e_pad, npad):
    D = tab.shape[1]
    ept2 = 2 * e_pad // NW
    ch2 = ept2 // CHUNK
    chh = ch2 // 2
    npt = npad // NS
    mesh = plsc.VectorSubcoreMesh(core_axis_name="c", subcore_axis_name="s")

    @functools.partial(
        pl.kernel,
        out_type=(
            jax.ShapeDtypeStruct((e_pad, D), jnp.float32),
            jax.ShapeDtypeStruct((e_pad, D), jnp.float32),
        ),
        mesh=mesh,
        scratch_types=[
            pltpu.VMEM_SHARED((npad, D), jnp.float32),
            pltpu.VMEM((chh, CHUNK), jnp.int32),
            pltpu.VMEM((CHUNK, D), jnp.float32),
            pltpu.VMEM((CHUNK, D), jnp.float32),
            pltpu.SemaphoreType.DMA,
            pltpu.SemaphoreType.DMA,
        ],
        compiler_params=_sc_compiler_params(),
    )
    def k(tab_hbm, ix_hbm, g1_hbm, g2_hbm, spm, ix_v, ba, bb, sa, sb):
        cid = lax.axis_index("c")
        sid = lax.axis_index("s")
        wid = cid * NS + sid
        pltpu.sync_copy(tab_hbm.at[pl.ds(cid * npad + sid * npt, npt)],
                        spm.at[pl.ds(sid * npt, npt)])
        plsc.subcore_barrier()
        lbase = sid * ept2

        def issue(jj, buf, s):
            pltpu.async_copy(spm.at[ix_v.at[jj]], buf, s)

        def finish(jj, half, buf, s):
            pltpu.make_async_copy(spm.at[ix_v.at[jj]], buf, s).wait()
            dst = pl.ds(lbase + (half * chh + jj) * CHUNK, CHUNK)

            @pl.when(cid == 0)
            def _():
                pltpu.sync_copy(buf, g1_hbm.at[dst])

            @pl.when(cid == 1)
            def _():
                pltpu.sync_copy(buf, g2_hbm.at[dst])

        for half in range(2):
            pltpu.sync_copy(
                ix_hbm.at[pl.ds(wid * ch2 + half * chh, chh)], ix_v)
            issue(0, ba, sa)

            @pl.loop(0, chh, step=2)
            def _(j):
                issue(j + 1, bb, sb)
                finish(j, half, ba, sa)

                @pl.when(j + 2 < chh)
                def _():
                    issue(j + 2, ba, sa)

                finish(j + 1, half, bb, sb)

    return k(tab, idx2d)


def _unpack_bf16_pair(x):
    u = lax.bitcast_convert_type(x, jnp.uint32)
    lo = lax.bitcast_convert_type((u & 0xFFFF).astype(jnp.uint16),
                                  jnp.bfloat16)
    hi = lax.bitcast_convert_type((u >> 16).astype(jnp.uint16), jnp.bfloat16)
    return lo, hi


def _edge_mlp(g1, g2, w1r, We2, be2r, Wc1, bc1r, wc2r, bc2r, n_edges, blk):
    e_pad, D = g1.shape
    hlf = D // 2
    grid = (e_pad // blk,)

    def body(g1_ref, g2_ref, w1_ref, w2_ref, b2_ref, wc1_ref,
             bc1_ref, wc2_ref, bc2_ref, ef_ref, tr_ref):
        gv1 = g1_ref[...]
        gv2 = g2_ref[...]
        lo1, hi1 = _unpack_bf16_pair(gv1[:, :hlf])
        lo2, hi2 = _unpack_bf16_pair(gv2[:, :hlf])
        s = jnp.concatenate(
            [lo1.astype(jnp.float32) + lo2.astype(jnp.float32),
             hi1.astype(jnp.float32) + hi2.astype(jnp.float32)], axis=1)
        cdv = gv1[:, hlf:hlf + 16] - gv2[:, hlf:hlf + 16]
        radial = jnp.sum(cdv * cdv, axis=1, keepdims=True)
        ef = _silu((s + radial * w1_ref[...]).astype(jnp.bfloat16))
        edge_feat = _silu(
            (jnp.dot(ef, w2_ref[...].astype(jnp.bfloat16),
                     preferred_element_type=jnp.float32)
             + b2_ref[...]).astype(jnp.bfloat16))
        tt = _silu(
            (jnp.dot(edge_feat, wc1_ref[...].astype(jnp.bfloat16),
                     preferred_element_type=jnp.float32)
             + bc1_ref[...]).astype(jnp.bfloat16))
        t = jnp.dot(tt, wc2_ref[...].astype(jnp.bfloat16),
                    preferred_element_type=jnp.float32)[:, 0:1] + bc2_ref[...]
        scale = t / jnp.sqrt(radial + 1e-8)
        trans = cdv * scale
        eid = pl.program_id(0) * blk + lax.broadcasted_iota(
            jnp.int32, (blk, 1), 0)
        valid = eid < n_edges
        ef_ref[...] = jnp.where(valid, edge_feat.astype(jnp.float32), 0.0)
        tr_ref[...] = jnp.transpose(jnp.where(valid, trans, 0.0))

    const = pl.BlockSpec((1, D), lambda i: (0, 0))
    return pl.pallas_call(
        body,
        grid=grid,
        in_specs=[
            pl.BlockSpec((blk, D), lambda i: (i, 0)),
            pl.BlockSpec((blk, D), lambda i: (i, 0)),
            const,
            pl.BlockSpec((D, D), lambda i: (0, 0)),
            const,
            pl.BlockSpec((D, D), lambda i: (0, 0)),
            const,
            pl.BlockSpec((D, D), lambda i: (0, 0)),
            pl.BlockSpec((1, 1), lambda i: (0, 0)),
        ],
        out_specs=[
            pl.BlockSpec((blk, D), lambda i: (i, 0)),
            pl.BlockSpec((16, blk), lambda i: (0, i)),
        ],
        out_shape=(
            jax.ShapeDtypeStruct((e_pad, D), jnp.float32),
            jax.ShapeDtypeStruct((16, e_pad), jnp.float32),
        ),
        compiler_params=pltpu.CompilerParams(
            dimension_semantics=("parallel",)),
    )(g1, g2, w1r, We2, be2r, Wc1, bc1r, wc2r, bc2r)


def _sc_segment_sum_h(ef, rows2d, zh, npad):
    e_pad, D = ef.shape
    ept = e_pad // NW
    ch = ept // CHUNK
    npt = npad // NS
    mesh = plsc.VectorSubcoreMesh(core_axis_name="c", subcore_axis_name="s")

    @functools.partial(
        pl.kernel,
        out_type=jax.ShapeDtypeStruct((NC * npad, D), jnp.float32),
        mesh=mesh,
        scratch_types=[
            pltpu.VMEM_SHARED((npad, D), jnp.float32),
            pltpu.VMEM((ch, CHUNK), jnp.int32),
            pltpu.VMEM((CHUNK, D), jnp.float32),
            pltpu.VMEM((CHUNK, D), jnp.float32),
            pltpu.SemaphoreType.DMA,
            pltpu.SemaphoreType.DMA,
        ],
        compiler_params=_sc_compiler_params(),
    )
    def k(ef_hbm, ri_hbm, zh_hbm, ph_hbm, acc_h, idx_v, vha, vhb, sa, sb):
        cid = lax.axis_index("c")
        sid = lax.axis_index("s")
        wid = cid * NS + sid
        nslc = pl.ds(sid * npt, npt)
        pltpu.sync_copy(zh_hbm, acc_h.at[nslc])
        pltpu.sync_copy(ri_hbm.at[pl.ds(wid * ch, ch)], idx_v)
        plsc.subcore_barrier()
        base = wid * ept

        def src(jj):
            return ef_hbm.at[pl.ds(base + jj * CHUNK, CHUNK)]

        pltpu.async_copy(src(0), vha, sa)

        @pl.loop(0, ch, step=2)
        def _(j):
            pltpu.async_copy(src(j + 1), vhb, sb)
            pltpu.make_async_copy(src(j), vha, sa).wait()
            pltpu.sync_copy(vha, acc_h.at[idx_v.at[j]], add=True)

            @pl.when(j + 2 < ch)
            def _():
                pltpu.async_copy(src(j + 2), vha, sa)

            pltpu.make_async_copy(src(j + 1), vhb, sb).wait()
            pltpu.sync_copy(vhb, acc_h.at[idx_v.at[j + 1]], add=True)

        plsc.subcore_barrier()
        pltpu.sync_copy(acc_h.at[nslc],
                        ph_hbm.at[pl.ds(cid * npad + sid * npt, npt)])

    return k(ef, rows2d, zh)


def _sc_segment_sum_c(trt, rows2d, zc, npad):
    e_pad = trt.shape[1]
    ept = e_pad // NW
    ch = ept // CHUNK
    n4 = npad * 4
    mesh = plsc.VectorSubcoreMesh(core_axis_name="c", subcore_axis_name="s")

    @functools.partial(
        pl.kernel,
        out_type=jax.ShapeDtypeStruct((NW * n4,), jnp.float32),
        mesh=mesh,
        scratch_types=[
            pltpu.VMEM((n4,), jnp.float32),
            pltpu.VMEM((ch, CHUNK), jnp.int32),
            pltpu.VMEM((16, CHUNK), jnp.float32),
            pltpu.VMEM((16, CHUNK), jnp.float32),
            pltpu.SemaphoreType.DMA,
            pltpu.SemaphoreType.DMA,
        ],
        compiler_params=_sc_compiler_params(),
    )
    def k(tr_hbm, ri_hbm, zc_hbm, pc_hbm, acc_c, idx_v, vta, vtb, sa, sb):
        wid = lax.axis_index("c") * NS + lax.axis_index("s")
        pltpu.sync_copy(zc_hbm, acc_c)
        pltpu.sync_copy(ri_hbm.at[pl.ds(wid * ch, ch)], idx_v)
        base = wid * ept

        def src(jj):
            return tr_hbm.at[:, pl.ds(base + jj * CHUNK, CHUNK)]

        def accumulate(jj, vt_v):
            for sub in range(CHUNK // 16):
                en = idx_v[jj, pl.ds(sub * 16, 16)]
                for c in range(3):
                    v = vt_v[c, pl.ds(sub * 16, 16)]
                    plsc.addupdate_scatter(acc_c, [en * 4 + c], v)

        pltpu.async_copy(src(0), vta, sa)

        @pl.loop(0, ch, step=2)
        def _(j):
            pltpu.async_copy(src(j + 1), vtb, sb)
            pltpu.make_async_copy(src(j), vta, sa).wait()
            accumulate(j, vta)

            @pl.when(j + 2 < ch)
            def _():
                pltpu.async_copy(src(j + 2), vta, sa)

            pltpu.make_async_copy(src(j + 1), vtb, sb).wait()
            accumulate(j + 1, vtb)

        pltpu.sync_copy(acc_c, pc_hbm.at[pl.ds(wid * n4, n4)])

    return k(trt, rows2d, zc)


def _node_mlp(h, c4mat, phs, pcmats, Wn1a, Wn1b, bn1r, Wn2, bn2r, npad):
    N, D = h.shape
    rows4 = c4mat.shape[0]
    ns = len(phs)

    def body(h_ref, c_ref, *refs):
        ph_refs = refs[:ns]
        pc_refs = refs[ns:2 * ns]
        wa_ref, wb_ref, b1_ref, w2_ref, b2_ref, ho_ref, co_ref = refs[2 * ns:]
        hh = h_ref[...]
        agg = sum(pr[:N, :] + pr[npad:npad + N, :] for pr in ph_refs)
        m1 = _silu(
            jnp.dot(hh, wa_ref[...], preferred_element_type=jnp.float32)
            + jnp.dot(agg, wb_ref[...], preferred_element_type=jnp.float32)
            + b1_ref[...])
        m = jnp.dot(m1, w2_ref[...], preferred_element_type=jnp.float32)
        ho_ref[...] = hh + m + b2_ref[...]
        aggc = sum(jnp.sum(pr[...], axis=0) for pr in pc_refs)
        co_ref[...] = c_ref[...] + aggc

    return pl.pallas_call(
        body,
        out_shape=(
            jax.ShapeDtypeStruct((N, D), jnp.float32),
            jax.ShapeDtypeStruct((rows4, 128), jnp.float32),
        ),
    )(h, c4mat, *phs, *pcmats, Wn1a, Wn1b, bn1r, Wn2, bn2r)


def kernel(h, edge_index, coord, edge_attr,
           We1, be1, We2, be2, Wn1, bn1, Wn2, bn2, Wc1, bc1, Wc2, bc2):
    del edge_attr
    N, D = h.shape
    E = edge_index.shape[1]
    tile_edges = NW * CHUNK * 8
    e_pad = ((E + tile_edges - 1) // tile_edges) * tile_edges
    npad = ((N + NS * 8 - 1) // (NS * 8)) * (NS * 8)

    row = edge_index[0].astype(jnp.int32)
    col = edge_index[1].astype(jnp.int32)
    rows2d = jnp.pad(row, (0, e_pad - E)).reshape(e_pad // CHUNK, CHUNK)
    cols2d = jnp.pad(col, (0, e_pad - E)).reshape(e_pad // CHUNK, CHUNK)

    c4flat = jnp.pad(coord, ((0, npad - N), (0, 1))).reshape(-1)
    We1a = We1[:D]
    We1b = We1[D:2 * D]
    w1r = We1[2 * D].reshape(1, D)
    be1r = be1.reshape(1, D)
    be2r = be2.reshape(1, D)
    bc1r = bc1.reshape(1, D)
    wc2r = jnp.pad(Wc2, ((0, 0), (0, D - 1)))
    bc2r = bc2.reshape(1, 1)
    bn1r = bn1.reshape(1, D)
    bn2r = bn2.reshape(1, D)
    Wn1a = Wn1[:D]
    Wn1b = Wn1[D:]

    c16 = jnp.pad(coord, ((0, 0), (0, 16 - coord.shape[1])))
    tab = _build_tables(h, c16, We1a, We1b, be1r, npad)
    zh = jnp.zeros((npad // NS, D), jnp.float32)
    zc = jnp.zeros((npad * 4,), jnp.float32)

    S = 1
    es = e_pad // S
    rps = es // CHUNK
    phs, pcs = [], []
    for s in range(S):
        r2d = lax.slice_in_dim(rows2d, s * rps, (s + 1) * rps, axis=0)
        c2d = lax.slice_in_dim(cols2d, s * rps, (s + 1) * rps, axis=0)
        idx2d = jnp.concatenate([r2d, c2d], axis=0)
        g1, g2 = _sc_gather(tab, idx2d, es, npad)
        nval = max(0, min(es, E - s * es))
        ef, trt = _edge_mlp(g1, g2, w1r, We2, be2r, Wc1,
                            bc1r, wc2r, bc2r, nval, 4096)
        phs.append(_sc_segment_sum_h(ef, r2d, zh, npad))
        pcs.append(_sc_segment_sum_c(trt, r2d, zc, npad))

    rows4 = npad * 4 // 128
    pcmats = [pc.reshape(NW, rows4, 128) for pc in pcs]
    c4mat = c4flat.reshape(rows4, 128)
    h_out, co_mat = _node_mlp(h, c4mat, phs, pcmats, Wn1a, Wn1b, bn1r,
                              Wn2, bn2r, npad)
    coord_out = co_mat.reshape(npad, 4)[:N, :3]
    return (h_out, coord_out)
